# Initial kernel scaffold; baseline (speedup 1.0000x reference)
import jax
import jax.numpy as jnp
from jax.experimental import pallas as pl


def kernel(x, edge_attr, edge_index, node_FVattr, edge_FVattr, params):
    raise NotImplementedError("write your pallas kernel here")


# speedup vs baseline: 3.4088x; 3.4088x over previous
import functools
import jax
import jax.numpy as jnp
from jax import lax
from jax.experimental import pallas as pl
from jax.experimental.pallas import tpu as pltpu
from jax.experimental.pallas import tpu_sc as plsc

NC = 2
NS = 16
NW = NC * NS

BE = 2560
BN = 2000
KCH = 400
KSC = 200


def _ln_fused(y, g, b):
    m = jnp.mean(y, axis=-1, keepdims=True)
    d = y - m
    v = jnp.mean(d * d, axis=-1, keepdims=True)
    return d * lax.rsqrt(v + 1e-5) * g + b


def _enc_body(x_ref, w1_ref, b1_ref, w2_ref, b2_ref, g_ref, be_ref, o_ref):
    a = jnp.maximum(
        jnp.dot(x_ref[...], w1_ref[...], preferred_element_type=jnp.float32)
        + b1_ref[...], 0.0)
    y = jnp.dot(a, w2_ref[...], preferred_element_type=jnp.float32) + b2_ref[...]
    o_ref[...] = _ln_fused(y, g_ref[...], be_ref[...])


def _encode(xin, W1, b1, W2, b2, g, be, BR):
    R, Din = xin.shape
    H = W2.shape[1]
    return pl.pallas_call(
        _enc_body,
        grid=(R // BR,),
        in_specs=[
            pl.BlockSpec((BR, Din), lambda i: (i, 0)),
            pl.BlockSpec((Din, H), lambda i: (0, 0)),
            pl.BlockSpec((H,), lambda i: (0,)),
            pl.BlockSpec((H, H), lambda i: (0, 0)),
            pl.BlockSpec((H,), lambda i: (0,)),
            pl.BlockSpec((H,), lambda i: (0,)),
            pl.BlockSpec((H,), lambda i: (0,)),
        ],
        out_specs=pl.BlockSpec((BR, H), lambda i: (i, 0)),
        out_shape=jax.ShapeDtypeStruct((R, H), jnp.float32),
    )(xin, W1, b1, W2, b2, g, be)


def _proj_body(h_ref, nfv_ref, ah_ref, af_ref, bh_ref, bf_ref, pa_ref, pb_ref):
    h = h_ref[...]
    nfv = nfv_ref[...]
    pa_ref[...] = jnp.dot(h, ah_ref[...], preferred_element_type=jnp.float32) + nfv * af_ref[...]
    pb_ref[...] = jnp.dot(h, bh_ref[...], preferred_element_type=jnp.float32) + nfv * bf_ref[...]


def _project(h_node, nfv, Ah, Af, Bh, Bf):
    Nn, H = h_node.shape
    out = jax.ShapeDtypeStruct((Nn, H), jnp.float32)
    return pl.pallas_call(
        _proj_body,
        grid=(Nn // BN,),
        in_specs=[
            pl.BlockSpec((BN, H), lambda i: (i, 0)),
            pl.BlockSpec((BN, 1), lambda i: (i, 0)),
            pl.BlockSpec((H, H), lambda i: (0, 0)),
            pl.BlockSpec((1, H), lambda i: (0, 0)),
            pl.BlockSpec((H, H), lambda i: (0, 0)),
            pl.BlockSpec((1, H), lambda i: (0, 0)),
        ],
        out_specs=[
            pl.BlockSpec((BN, H), lambda i: (i, 0)),
            pl.BlockSpec((BN, H), lambda i: (i, 0)),
        ],
        out_shape=[out, out],
    )(h_node, nfv, Ah, Af, Bh, Bf)


def _edge_body(he_ref, ga_ref, gb_ref, efv_ref, ch_ref, cf_ref, b1_ref,
               w2_ref, b2_ref, g_ref, be_ref, o_ref):
    he = he_ref[...]
    z = (ga_ref[...] + gb_ref[...]
         + jnp.dot(he, ch_ref[...], preferred_element_type=jnp.float32)
         + jnp.dot(efv_ref[...], cf_ref[...], preferred_element_type=jnp.float32)
         + b1_ref[...])
    a = jnp.maximum(z, 0.0)
    y = jnp.dot(a, w2_ref[...], preferred_element_type=jnp.float32) + b2_ref[...]
    o_ref[...] = he + _ln_fused(y, g_ref[...], be_ref[...])


def _edge_update(h_edge, GA, GB, efv, Ch, Cf, b1, W2, b2, g, be):
    E, H = h_edge.shape
    F = efv.shape[1]
    return pl.pallas_call(
        _edge_body,
        grid=(E // BE,),
        in_specs=[
            pl.BlockSpec((BE, H), lambda i: (i, 0)),
            pl.BlockSpec((BE, H), lambda i: (i, 0)),
            pl.BlockSpec((BE, H), lambda i: (i, 0)),
            pl.BlockSpec((BE, F), lambda i: (i, 0)),
            pl.BlockSpec((H, H), lambda i: (0, 0)),
            pl.BlockSpec((F, H), lambda i: (0, 0)),
            pl.BlockSpec((H,), lambda i: (0,)),
            pl.BlockSpec((H, H), lambda i: (0, 0)),
            pl.BlockSpec((H,), lambda i: (0,)),
            pl.BlockSpec((H,), lambda i: (0,)),
            pl.BlockSpec((H,), lambda i: (0,)),
        ],
        out_specs=pl.BlockSpec((BE, H), lambda i: (i, 0)),
        out_shape=jax.ShapeDtypeStruct((E, H), jnp.float32),
    )(h_edge, GA, GB, efv, Ch, Cf, b1, W2, b2, g, be)


def _node_body(h_ref, nfv_ref, a0_ref, a1_ref, nh_ref, nf_ref, na_ref,
               b1_ref, w2_ref, b2_ref, g_ref, be_ref, o_ref):
    h = h_ref[...]
    agg = a0_ref[...] + a1_ref[...]
    z = (jnp.dot(h, nh_ref[...], preferred_element_type=jnp.float32)
         + nfv_ref[...] * nf_ref[...]
         + jnp.dot(agg, na_ref[...], preferred_element_type=jnp.float32)
         + b1_ref[...])
    a = jnp.maximum(z, 0.0)
    y = jnp.dot(a, w2_ref[...], preferred_element_type=jnp.float32) + b2_ref[...]
    o_ref[...] = h + _ln_fused(y, g_ref[...], be_ref[...])


def _node_update(h_node, nfv, agg0, agg1, Nh, Nf, Na, b1, W2, b2, g, be):
    Nn, H = h_node.shape
    return pl.pallas_call(
        _node_body,
        grid=(Nn // BN,),
        in_specs=[
            pl.BlockSpec((BN, H), lambda i: (i, 0)),
            pl.BlockSpec((BN, 1), lambda i: (i, 0)),
            pl.BlockSpec((BN, H), lambda i: (i, 0)),
            pl.BlockSpec((BN, H), lambda i: (i, 0)),
            pl.BlockSpec((H, H), lambda i: (0, 0)),
            pl.BlockSpec((1, H), lambda i: (0, 0)),
            pl.BlockSpec((H, H), lambda i: (0, 0)),
            pl.BlockSpec((H,), lambda i: (0,)),
            pl.BlockSpec((H, H), lambda i: (0, 0)),
            pl.BlockSpec((H,), lambda i: (0,)),
            pl.BlockSpec((H,), lambda i: (0,)),
            pl.BlockSpec((H,), lambda i: (0,)),
        ],
        out_specs=pl.BlockSpec((BN, H), lambda i: (i, 0)),
        out_shape=jax.ShapeDtypeStruct((Nn, H), jnp.float32),
    )(h_node, nfv, agg0, agg1, Nh, Nf, Na, b1, W2, b2, g, be)


def _dec_body(h_ref, nfv_ref, w1h_ref, w1f_ref, b1_ref, w2_ref, b2_ref, o_ref):
    z = (jnp.dot(h_ref[...], w1h_ref[...], preferred_element_type=jnp.float32)
         + nfv_ref[...] * w1f_ref[...] + b1_ref[...])
    a = jnp.maximum(z, 0.0)
    o_ref[...] = jnp.dot(a, w2_ref[...], preferred_element_type=jnp.float32) + b2_ref[...]


def _decode(h_node, nfv, W1h, W1f, b1, W2, b2):
    Nn, H = h_node.shape
    D1 = W1h.shape[1]
    DO = W2.shape[1]
    return pl.pallas_call(
        _dec_body,
        grid=(Nn // BN,),
        in_specs=[
            pl.BlockSpec((BN, H), lambda i: (i, 0)),
            pl.BlockSpec((BN, 1), lambda i: (i, 0)),
            pl.BlockSpec((H, D1), lambda i: (0, 0)),
            pl.BlockSpec((1, D1), lambda i: (0, 0)),
            pl.BlockSpec((D1,), lambda i: (0,)),
            pl.BlockSpec((D1, DO), lambda i: (0, 0)),
            pl.BlockSpec((DO,), lambda i: (0,)),
        ],
        out_specs=pl.BlockSpec((BN, DO), lambda i: (i, 0)),
        out_shape=jax.ShapeDtypeStruct((Nn, DO), jnp.float32),
    )(h_node, nfv, W1h, W1f, b1, W2, b2)


def _sc_gather(table, idx):
    Erows = idx.shape[0]
    Hd = table.shape[1]
    EW = Erows // NW
    nchunk = EW // KCH
    mesh = plsc.VectorSubcoreMesh(core_axis_name="c", subcore_axis_name="s")

    def body(tbl_hbm, idx_hbm, out_hbm, idx_v, rows_v, sem):
        wid = lax.axis_index("s") * NC + lax.axis_index("c")
        base = wid * EW

        def chunk(c, carry):
            off = base + c * KCH
            pltpu.sync_copy(idx_hbm.at[pl.ds(off, KCH)], idx_v)
            pltpu.async_copy(tbl_hbm.at[idx_v], rows_v, sem).wait()
            pltpu.sync_copy(rows_v, out_hbm.at[pl.ds(off, KCH)])
            return carry

        lax.fori_loop(0, nchunk, chunk, 0)

    return pl.kernel(
        body,
        out_type=jax.ShapeDtypeStruct((Erows, Hd), jnp.float32),
        mesh=mesh,
        scratch_types=[
            pltpu.VMEM((KCH,), jnp.int32),
            pltpu.VMEM((KCH, Hd), jnp.float32),
            pltpu.SemaphoreType.DMA,
        ],
    )(table, idx)


def _sc_scatter(rows, idx, zeros):
    Erows, Hd = rows.shape
    Nn = zeros.shape[0]
    EW = Erows // NW
    nchunk = EW // KCH
    mesh = plsc.VectorSubcoreMesh(core_axis_name="c", subcore_axis_name="s")

    def body(rows_hbm, idx_hbm, zeros_hbm, out_hbm, idx_v, rows_v, accum, sem):
        cid = lax.axis_index("c")
        sid = lax.axis_index("s")
        wid = sid * NC + cid

        @pl.when(sid == 0)
        def _():
            pltpu.sync_copy(zeros_hbm, accum)

        plsc.subcore_barrier()
        base = wid * EW

        def chunk(c, carry):
            off = base + c * KSC
            pltpu.sync_copy(idx_hbm.at[pl.ds(off, KSC)], idx_v)
            pltpu.async_copy(rows_hbm.at[pl.ds(off, KSC)], rows_v, sem).wait()
            pltpu.sync_copy(rows_v, accum.at[idx_v], add=True)
            return carry

        lax.fori_loop(0, nchunk, chunk, 0)
        plsc.subcore_barrier()

        @pl.when(sid == 0)
        def _():
            pltpu.sync_copy(accum, out_hbm.at[cid])

    return pl.kernel(
        body,
        out_type=jax.ShapeDtypeStruct((NC, Nn, Hd), jnp.float32),
        mesh=mesh,
        scratch_types=[
            pltpu.VMEM((KSC,), jnp.int32),
            pltpu.VMEM((KSC, Hd), jnp.float32),
            pltpu.VMEM_SHARED((Nn, Hd), jnp.float32),
            pltpu.SemaphoreType.DMA,
        ],
    )(rows, idx, zeros)


def kernel(x, edge_attr, edge_index, node_FVattr, edge_FVattr, params):
    H = 128
    Nn = x.shape[0]
    src = edge_index[0]
    dst = edge_index[1]
    nfv = node_FVattr
    efv = edge_FVattr
    zeros = jnp.zeros((Nn, H), jnp.float32)

    x_in = jnp.concatenate([x, nfv], axis=1)
    e_in = jnp.concatenate([edge_attr, efv], axis=1)
    (We1, be1), (We2, be2) = params['enc_node_mlp']
    gn, bn = params['enc_node_ln']
    h_node = _encode(x_in, We1, be1, We2, be2, gn, bn, BN)
    (Wf1, bf1), (Wf2, bf2) = params['enc_edge_mlp']
    ge, bse = params['enc_edge_ln']
    h_edge = _encode(e_in, Wf1, bf1, Wf2, bf2, ge, bse, BE)

    def stk(f):
        return jnp.stack([f(c) for c in params['convs']])

    cw = {
        'Ah': stk(lambda c: c['edge_mlp'][0][0][0:H]),
        'Af': stk(lambda c: c['edge_mlp'][0][0][H:H + 1]),
        'Bh': stk(lambda c: c['edge_mlp'][0][0][H + 1:2 * H + 1]),
        'Bf': stk(lambda c: c['edge_mlp'][0][0][2 * H + 1:2 * H + 2]),
        'Ch': stk(lambda c: c['edge_mlp'][0][0][2 * H + 2:3 * H + 2]),
        'Cf': stk(lambda c: c['edge_mlp'][0][0][3 * H + 2:]),
        'eb1': stk(lambda c: c['edge_mlp'][0][1]),
        'eW2': stk(lambda c: c['edge_mlp'][1][0]),
        'eb2': stk(lambda c: c['edge_mlp'][1][1]),
        'eg': stk(lambda c: c['edge_ln'][0]),
        'ebeta': stk(lambda c: c['edge_ln'][1]),
        'Nh': stk(lambda c: c['node_mlp'][0][0][0:H]),
        'Nf': stk(lambda c: c['node_mlp'][0][0][H:H + 1]),
        'Na': stk(lambda c: c['node_mlp'][0][0][H + 1:]),
        'nb1': stk(lambda c: c['node_mlp'][0][1]),
        'nW2': stk(lambda c: c['node_mlp'][1][0]),
        'nb2': stk(lambda c: c['node_mlp'][1][1]),
        'ng': stk(lambda c: c['node_ln'][0]),
        'nbeta': stk(lambda c: c['node_ln'][1]),
    }

    def conv_step(carry, w):
        h_node, h_edge = carry
        PA, PB = _project(h_node, nfv, w['Ah'], w['Af'], w['Bh'], w['Bf'])
        GA = _sc_gather(PA, src)
        GB = _sc_gather(PB, dst)
        h_edge = _edge_update(h_edge, GA, GB, efv, w['Ch'], w['Cf'],
                              w['eb1'], w['eW2'], w['eb2'], w['eg'], w['ebeta'])
        aggs = _sc_scatter(h_edge, dst, zeros)
        h_node = _node_update(h_node, nfv, aggs[0], aggs[1], w['Nh'], w['Nf'],
                              w['Na'], w['nb1'], w['nW2'], w['nb2'],
                              w['ng'], w['nbeta'])
        return (h_node, h_edge), None

    (h_node, h_edge), _ = lax.scan(conv_step, (h_node, h_edge), cw)

    (Wd1, bd1), (Wd2, bd2) = params['dec_mlp']
    return _decode(h_node, nfv, Wd1[0:H], Wd1[H:H + 1], bd1, Wd2, bd2)


# speedup vs baseline: 3.6510x; 1.0711x over previous
import functools
import jax
import jax.numpy as jnp
from jax import lax
from jax.experimental import pallas as pl
from jax.experimental.pallas import tpu as pltpu
from jax.experimental.pallas import tpu_sc as plsc

NC = 2
NS = 16
NW = NC * NS

BE = 2560
BN = 2000
KCH = 200
KSC = 200


def _ln_fused(y, g, b):
    m = jnp.mean(y, axis=-1, keepdims=True)
    d = y - m
    v = jnp.mean(d * d, axis=-1, keepdims=True)
    return d * lax.rsqrt(v + 1e-5) * g + b


def _enc_body(x_ref, w1_ref, b1_ref, w2_ref, b2_ref, g_ref, be_ref, o_ref):
    a = jnp.maximum(
        jnp.dot(x_ref[...], w1_ref[...], preferred_element_type=jnp.float32)
        + b1_ref[...], 0.0)
    y = jnp.dot(a, w2_ref[...], preferred_element_type=jnp.float32) + b2_ref[...]
    o_ref[...] = _ln_fused(y, g_ref[...], be_ref[...])


def _encode(xin, W1, b1, W2, b2, g, be, BR):
    R, Din = xin.shape
    H = W2.shape[1]
    return pl.pallas_call(
        _enc_body,
        grid=(R // BR,),
        in_specs=[
            pl.BlockSpec((BR, Din), lambda i: (i, 0)),
            pl.BlockSpec((Din, H), lambda i: (0, 0)),
            pl.BlockSpec((H,), lambda i: (0,)),
            pl.BlockSpec((H, H), lambda i: (0, 0)),
            pl.BlockSpec((H,), lambda i: (0,)),
            pl.BlockSpec((H,), lambda i: (0,)),
            pl.BlockSpec((H,), lambda i: (0,)),
        ],
        out_specs=pl.BlockSpec((BR, H), lambda i: (i, 0)),
        out_shape=jax.ShapeDtypeStruct((R, H), jnp.float32),
    )(xin, W1, b1, W2, b2, g, be)


def _proj_body(h_ref, nfv_ref, ah_ref, af_ref, bh_ref, bf_ref, pa_ref, pb_ref):
    h = h_ref[...]
    nfv = nfv_ref[...]
    pa_ref[...] = jnp.dot(h, ah_ref[...], preferred_element_type=jnp.float32) + nfv * af_ref[...]
    pb_ref[...] = jnp.dot(h, bh_ref[...], preferred_element_type=jnp.float32) + nfv * bf_ref[...]


def _project(h_node, nfv, Ah, Af, Bh, Bf):
    Nn, H = h_node.shape
    out = jax.ShapeDtypeStruct((Nn, H), jnp.float32)
    return pl.pallas_call(
        _proj_body,
        grid=(Nn // BN,),
        in_specs=[
            pl.BlockSpec((BN, H), lambda i: (i, 0)),
            pl.BlockSpec((BN, 1), lambda i: (i, 0)),
            pl.BlockSpec((H, H), lambda i: (0, 0)),
            pl.BlockSpec((1, H), lambda i: (0, 0)),
            pl.BlockSpec((H, H), lambda i: (0, 0)),
            pl.BlockSpec((1, H), lambda i: (0, 0)),
        ],
        out_specs=[
            pl.BlockSpec((BN, H), lambda i: (i, 0)),
            pl.BlockSpec((BN, H), lambda i: (i, 0)),
        ],
        out_shape=[out, out],
    )(h_node, nfv, Ah, Af, Bh, Bf)


def _edge_body(he_ref, ga_ref, efv_ref, ch_ref, cf_ref, b1_ref,
               w2_ref, b2_ref, g_ref, be_ref, o_ref):
    he = he_ref[...]
    z = (ga_ref[...]
         + jnp.dot(he, ch_ref[...], preferred_element_type=jnp.float32)
         + jnp.dot(efv_ref[...], cf_ref[...], preferred_element_type=jnp.float32)
         + b1_ref[...])
    a = jnp.maximum(z, 0.0)
    y = jnp.dot(a, w2_ref[...], preferred_element_type=jnp.float32) + b2_ref[...]
    o_ref[...] = he + _ln_fused(y, g_ref[...], be_ref[...])


def _edge_update(h_edge, G, efv, Ch, Cf, b1, W2, b2, g, be):
    E, H = h_edge.shape
    F = efv.shape[1]
    return pl.pallas_call(
        _edge_body,
        grid=(E // BE,),
        in_specs=[
            pl.BlockSpec((BE, H), lambda i: (i, 0)),
            pl.BlockSpec((BE, H), lambda i: (i, 0)),
            pl.BlockSpec((BE, F), lambda i: (i, 0)),
            pl.BlockSpec((H, H), lambda i: (0, 0)),
            pl.BlockSpec((F, H), lambda i: (0, 0)),
            pl.BlockSpec((H,), lambda i: (0,)),
            pl.BlockSpec((H, H), lambda i: (0, 0)),
            pl.BlockSpec((H,), lambda i: (0,)),
            pl.BlockSpec((H,), lambda i: (0,)),
            pl.BlockSpec((H,), lambda i: (0,)),
        ],
        out_specs=pl.BlockSpec((BE, H), lambda i: (i, 0)),
        out_shape=jax.ShapeDtypeStruct((E, H), jnp.float32),
    )(h_edge, G, efv, Ch, Cf, b1, W2, b2, g, be)


def _node_body(h_ref, nfv_ref, a0_ref, a1_ref, nh_ref, nf_ref, na_ref,
               b1_ref, w2_ref, b2_ref, g_ref, be_ref, o_ref):
    h = h_ref[...]
    agg = a0_ref[...] + a1_ref[...]
    z = (jnp.dot(h, nh_ref[...], preferred_element_type=jnp.float32)
         + nfv_ref[...] * nf_ref[...]
         + jnp.dot(agg, na_ref[...], preferred_element_type=jnp.float32)
         + b1_ref[...])
    a = jnp.maximum(z, 0.0)
    y = jnp.dot(a, w2_ref[...], preferred_element_type=jnp.float32) + b2_ref[...]
    o_ref[...] = h + _ln_fused(y, g_ref[...], be_ref[...])


def _node_update(h_node, nfv, agg0, agg1, Nh, Nf, Na, b1, W2, b2, g, be):
    Nn, H = h_node.shape
    return pl.pallas_call(
        _node_body,
        grid=(Nn // BN,),
        in_specs=[
            pl.BlockSpec((BN, H), lambda i: (i, 0)),
            pl.BlockSpec((BN, 1), lambda i: (i, 0)),
            pl.BlockSpec((BN, H), lambda i: (i, 0)),
            pl.BlockSpec((BN, H), lambda i: (i, 0)),
            pl.BlockSpec((H, H), lambda i: (0, 0)),
            pl.BlockSpec((1, H), lambda i: (0, 0)),
            pl.BlockSpec((H, H), lambda i: (0, 0)),
            pl.BlockSpec((H,), lambda i: (0,)),
            pl.BlockSpec((H, H), lambda i: (0, 0)),
            pl.BlockSpec((H,), lambda i: (0,)),
            pl.BlockSpec((H,), lambda i: (0,)),
            pl.BlockSpec((H,), lambda i: (0,)),
        ],
        out_specs=pl.BlockSpec((BN, H), lambda i: (i, 0)),
        out_shape=jax.ShapeDtypeStruct((Nn, H), jnp.float32),
    )(h_node, nfv, agg0, agg1, Nh, Nf, Na, b1, W2, b2, g, be)


def _dec_body(h_ref, nfv_ref, w1h_ref, w1f_ref, b1_ref, w2_ref, b2_ref, o_ref):
    z = (jnp.dot(h_ref[...], w1h_ref[...], preferred_element_type=jnp.float32)
         + nfv_ref[...] * w1f_ref[...] + b1_ref[...])
    a = jnp.maximum(z, 0.0)
    o_ref[...] = jnp.dot(a, w2_ref[...], preferred_element_type=jnp.float32) + b2_ref[...]


def _decode(h_node, nfv, W1h, W1f, b1, W2, b2):
    Nn, H = h_node.shape
    D1 = W1h.shape[1]
    DO = W2.shape[1]
    return pl.pallas_call(
        _dec_body,
        grid=(Nn // BN,),
        in_specs=[
            pl.BlockSpec((BN, H), lambda i: (i, 0)),
            pl.BlockSpec((BN, 1), lambda i: (i, 0)),
            pl.BlockSpec((H, D1), lambda i: (0, 0)),
            pl.BlockSpec((1, D1), lambda i: (0, 0)),
            pl.BlockSpec((D1,), lambda i: (0,)),
            pl.BlockSpec((D1, DO), lambda i: (0, 0)),
            pl.BlockSpec((DO,), lambda i: (0,)),
        ],
        out_specs=pl.BlockSpec((BN, DO), lambda i: (i, 0)),
        out_shape=jax.ShapeDtypeStruct((Nn, DO), jnp.float32),
    )(h_node, nfv, W1h, W1f, b1, W2, b2)


def _sc_gather(tableA, tableB, idxA, idxB):
    Erows = idxA.shape[0]
    Hd = tableA.shape[1]
    EW = Erows // NW
    K = KCH
    npair = EW // (2 * K)
    mesh = plsc.VectorSubcoreMesh(core_axis_name="c", subcore_axis_name="s")

    def body(ta_hbm, tb_hbm, idxa_hbm, idxb_hbm, out_hbm,
             idxa_v, idxb_v, a0, b0, a1, b1, sa0, sb0, sa1, sb1):
        wid = lax.axis_index("s") * NC + lax.axis_index("c")
        base = wid * EW
        pltpu.sync_copy(idxa_hbm.at[pl.ds(base, EW)], idxa_v)
        pltpu.sync_copy(idxb_hbm.at[pl.ds(base, EW)], idxb_v)

        def addrows(dst, srcb):
            def row(r, carry):
                for j in range(Hd // 16):
                    s = pl.ds(j * 16, 16)
                    dst[r, s] = dst[r, s] + srcb[r, s]
                return carry
            lax.fori_loop(0, K, row, 0)

        def pair(g, carry):
            c0 = 2 * g * K
            c1 = c0 + K
            h0a = pltpu.async_copy(ta_hbm.at[idxa_v.at[pl.ds(c0, K)]], a0, sa0)
            h0b = pltpu.async_copy(tb_hbm.at[idxb_v.at[pl.ds(c0, K)]], b0, sb0)
            h1a = pltpu.async_copy(ta_hbm.at[idxa_v.at[pl.ds(c1, K)]], a1, sa1)
            h1b = pltpu.async_copy(tb_hbm.at[idxb_v.at[pl.ds(c1, K)]], b1, sb1)
            h0a.wait()
            h0b.wait()
            addrows(a0, b0)
            pltpu.sync_copy(a0, out_hbm.at[pl.ds(base + c0, K)])
            h1a.wait()
            h1b.wait()
            addrows(a1, b1)
            pltpu.sync_copy(a1, out_hbm.at[pl.ds(base + c1, K)])
            return carry

        lax.fori_loop(0, npair, pair, 0)

    return pl.kernel(
        body,
        out_type=jax.ShapeDtypeStruct((Erows, Hd), jnp.float32),
        mesh=mesh,
        scratch_types=[
            pltpu.VMEM((EW,), jnp.int32),
            pltpu.VMEM((EW,), jnp.int32),
            pltpu.VMEM((K, Hd), jnp.float32),
            pltpu.VMEM((K, Hd), jnp.float32),
            pltpu.VMEM((K, Hd), jnp.float32),
            pltpu.VMEM((K, Hd), jnp.float32),
            pltpu.SemaphoreType.DMA,
            pltpu.SemaphoreType.DMA,
            pltpu.SemaphoreType.DMA,
            pltpu.SemaphoreType.DMA,
        ],
    )(tableA, tableB, idxA, idxB)


def _sc_scatter(rows, idx, zeros):
    Erows, Hd = rows.shape
    Nn = zeros.shape[0]
    EW = Erows // NW
    nchunk = EW // KCH
    mesh = plsc.VectorSubcoreMesh(core_axis_name="c", subcore_axis_name="s")

    def body(rows_hbm, idx_hbm, zeros_hbm, out_hbm, idx_v, rows_v, accum, sem):
        cid = lax.axis_index("c")
        sid = lax.axis_index("s")
        wid = sid * NC + cid

        @pl.when(sid == 0)
        def _():
            pltpu.sync_copy(zeros_hbm, accum)

        plsc.subcore_barrier()
        base = wid * EW

        def chunk(c, carry):
            off = base + c * KSC
            pltpu.sync_copy(idx_hbm.at[pl.ds(off, KSC)], idx_v)
            pltpu.async_copy(rows_hbm.at[pl.ds(off, KSC)], rows_v, sem).wait()
            pltpu.sync_copy(rows_v, accum.at[idx_v], add=True)
            return carry

        lax.fori_loop(0, nchunk, chunk, 0)
        plsc.subcore_barrier()

        @pl.when(sid == 0)
        def _():
            pltpu.sync_copy(accum, out_hbm.at[cid])

    return pl.kernel(
        body,
        out_type=jax.ShapeDtypeStruct((NC, Nn, Hd), jnp.float32),
        mesh=mesh,
        scratch_types=[
            pltpu.VMEM((KSC,), jnp.int32),
            pltpu.VMEM((KSC, Hd), jnp.float32),
            pltpu.VMEM_SHARED((Nn, Hd), jnp.float32),
            pltpu.SemaphoreType.DMA,
        ],
    )(rows, idx, zeros)


def kernel(x, edge_attr, edge_index, node_FVattr, edge_FVattr, params):
    H = 128
    Nn = x.shape[0]
    src = edge_index[0]
    dst = edge_index[1]
    nfv = node_FVattr
    efv = edge_FVattr
    zeros = jnp.zeros((Nn, H), jnp.float32)

    x_in = jnp.concatenate([x, nfv], axis=1)
    e_in = jnp.concatenate([edge_attr, efv], axis=1)
    (We1, be1), (We2, be2) = params['enc_node_mlp']
    gn, bn = params['enc_node_ln']
    h_node = _encode(x_in, We1, be1, We2, be2, gn, bn, BN)
    (Wf1, bf1), (Wf2, bf2) = params['enc_edge_mlp']
    ge, bse = params['enc_edge_ln']
    h_edge = _encode(e_in, Wf1, bf1, Wf2, bf2, ge, bse, BE)

    def stk(f):
        return jnp.stack([f(c) for c in params['convs']])

    cw = {
        'Ah': stk(lambda c: c['edge_mlp'][0][0][0:H]),
        'Af': stk(lambda c: c['edge_mlp'][0][0][H:H + 1]),
        'Bh': stk(lambda c: c['edge_mlp'][0][0][H + 1:2 * H + 1]),
        'Bf': stk(lambda c: c['edge_mlp'][0][0][2 * H + 1:2 * H + 2]),
        'Ch': stk(lambda c: c['edge_mlp'][0][0][2 * H + 2:3 * H + 2]),
        'Cf': stk(lambda c: c['edge_mlp'][0][0][3 * H + 2:]),
        'eb1': stk(lambda c: c['edge_mlp'][0][1]),
        'eW2': stk(lambda c: c['edge_mlp'][1][0]),
        'eb2': stk(lambda c: c['edge_mlp'][1][1]),
        'eg': stk(lambda c: c['edge_ln'][0]),
        'ebeta': stk(lambda c: c['edge_ln'][1]),
        'Nh': stk(lambda c: c['node_mlp'][0][0][0:H]),
        'Nf': stk(lambda c: c['node_mlp'][0][0][H:H + 1]),
        'Na': stk(lambda c: c['node_mlp'][0][0][H + 1:]),
        'nb1': stk(lambda c: c['node_mlp'][0][1]),
        'nW2': stk(lambda c: c['node_mlp'][1][0]),
        'nb2': stk(lambda c: c['node_mlp'][1][1]),
        'ng': stk(lambda c: c['node_ln'][0]),
        'nbeta': stk(lambda c: c['node_ln'][1]),
    }

    def conv_step(carry, w):
        h_node, h_edge = carry
        PA, PB = _project(h_node, nfv, w['Ah'], w['Af'], w['Bh'], w['Bf'])
        G = _sc_gather(PA, PB, src, dst)
        h_edge = _edge_update(h_edge, G, efv, w['Ch'], w['Cf'],
                              w['eb1'], w['eW2'], w['eb2'], w['eg'], w['ebeta'])
        aggs = _sc_scatter(h_edge, dst, zeros)
        h_node = _node_update(h_node, nfv, aggs[0], aggs[1], w['Nh'], w['Nf'],
                              w['Na'], w['nb1'], w['nW2'], w['nb2'],
                              w['ng'], w['nbeta'])
        return (h_node, h_edge), None

    (h_node, h_edge), _ = lax.scan(conv_step, (h_node, h_edge), cw)

    (Wd1, bd1), (Wd2, bd2) = params['dec_mlp']
    return _decode(h_node, nfv, Wd1[0:H], Wd1[H:H + 1], bd1, Wd2, bd2)


# speedup vs baseline: 3.9545x; 1.0831x over previous
import functools
import jax
import jax.numpy as jnp
import numpy as np
from jax import lax
from jax.experimental import pallas as pl
from jax.experimental.pallas import tpu as pltpu
from jax.experimental.pallas import tpu_sc as plsc

NC = 2
NS = 16
NW = NC * NS

BE = 2000
BN = 2000
KCH = 200
KSC = 200


def _ln_fused(y, g, b):
    m = jnp.mean(y, axis=-1, keepdims=True)
    d = y - m
    v = jnp.mean(d * d, axis=-1, keepdims=True)
    return d * lax.rsqrt(v + 1e-5) * g + b


def _enc_body(x_ref, w1_ref, b1_ref, w2_ref, b2_ref, g_ref, be_ref, o_ref):
    a = jnp.maximum(
        jnp.dot(x_ref[...], w1_ref[...], preferred_element_type=jnp.float32)
        + b1_ref[...], 0.0)
    y = jnp.dot(a, w2_ref[...], preferred_element_type=jnp.float32) + b2_ref[...]
    o_ref[...] = _ln_fused(y, g_ref[...], be_ref[...])


def _encode(xin, W1, b1, W2, b2, g, be, BR):
    R, Din = xin.shape
    H = W2.shape[1]
    return pl.pallas_call(
        _enc_body,
        grid=(R // BR,),
        in_specs=[
            pl.BlockSpec((BR, Din), lambda i: (i, 0)),
            pl.BlockSpec((Din, H), lambda i: (0, 0)),
            pl.BlockSpec((H,), lambda i: (0,)),
            pl.BlockSpec((H, H), lambda i: (0, 0)),
            pl.BlockSpec((H,), lambda i: (0,)),
            pl.BlockSpec((H,), lambda i: (0,)),
            pl.BlockSpec((H,), lambda i: (0,)),
        ],
        out_specs=pl.BlockSpec((BR, H), lambda i: (i, 0)),
        out_shape=jax.ShapeDtypeStruct((R, H), jnp.float32),
    )(xin, W1, b1, W2, b2, g, be)


def _proj_body(h_ref, nfv_ref, ah_ref, af_ref, bh_ref, bf_ref, pa_ref, pb_ref):
    h = h_ref[...]
    nfv = nfv_ref[...]
    pa_ref[...] = jnp.dot(h, ah_ref[...], preferred_element_type=jnp.float32) + nfv * af_ref[...]
    pb_ref[...] = jnp.dot(h, bh_ref[...], preferred_element_type=jnp.float32) + nfv * bf_ref[...]


def _project(h_node, nfv, Ah, Af, Bh, Bf):
    Nn, H = h_node.shape
    out = jax.ShapeDtypeStruct((Nn, H), jnp.float32)
    return pl.pallas_call(
        _proj_body,
        grid=(Nn // BN,),
        in_specs=[
            pl.BlockSpec((BN, H), lambda i: (i, 0)),
            pl.BlockSpec((BN, 1), lambda i: (i, 0)),
            pl.BlockSpec((H, H), lambda i: (0, 0)),
            pl.BlockSpec((1, H), lambda i: (0, 0)),
            pl.BlockSpec((H, H), lambda i: (0, 0)),
            pl.BlockSpec((1, H), lambda i: (0, 0)),
        ],
        out_specs=[
            pl.BlockSpec((BN, H), lambda i: (i, 0)),
            pl.BlockSpec((BN, H), lambda i: (i, 0)),
        ],
        out_shape=[out, out],
    )(h_node, nfv, Ah, Af, Bh, Bf)


def _edge_body(he_ref, ga_ref, efv_ref, ch_ref, cf_ref, b1_ref,
               w2_ref, b2_ref, g_ref, be_ref, o_ref):
    he = he_ref[...]
    z = (ga_ref[...]
         + jnp.dot(he, ch_ref[...], preferred_element_type=jnp.float32)
         + jnp.dot(efv_ref[...], cf_ref[...], preferred_element_type=jnp.float32)
         + b1_ref[...])
    a = jnp.maximum(z, 0.0)
    y = jnp.dot(a, w2_ref[...], preferred_element_type=jnp.float32) + b2_ref[...]
    o_ref[...] = he + _ln_fused(y, g_ref[...], be_ref[...])


def _edge_update(h_edge, G, efv, Ch, Cf, b1, W2, b2, g, be):
    E, H = h_edge.shape
    F = efv.shape[1]
    return pl.pallas_call(
        _edge_body,
        grid=(E // BE,),
        in_specs=[
            pl.BlockSpec((BE, H), lambda i: (i, 0)),
            pl.BlockSpec((BE, H), lambda i: (i, 0)),
            pl.BlockSpec((BE, F), lambda i: (i, 0)),
            pl.BlockSpec((H, H), lambda i: (0, 0)),
            pl.BlockSpec((F, H), lambda i: (0, 0)),
            pl.BlockSpec((H,), lambda i: (0,)),
            pl.BlockSpec((H, H), lambda i: (0, 0)),
            pl.BlockSpec((H,), lambda i: (0,)),
            pl.BlockSpec((H,), lambda i: (0,)),
            pl.BlockSpec((H,), lambda i: (0,)),
        ],
        out_specs=pl.BlockSpec((BE, H), lambda i: (i, 0)),
        out_shape=jax.ShapeDtypeStruct((E, H), jnp.float32),
    )(h_edge, G, efv, Ch, Cf, b1, W2, b2, g, be)


def _node_body(h_ref, nfv_ref, a00_ref, a01_ref, a10_ref, a11_ref,
               nh_ref, nf_ref, na_ref,
               b1_ref, w2_ref, b2_ref, g_ref, be_ref, o_ref):
    h = h_ref[...]
    agg = ((a00_ref[...] + a01_ref[...]) + (a10_ref[...] + a11_ref[...]))
    z = (jnp.dot(h, nh_ref[...], preferred_element_type=jnp.float32)
         + nfv_ref[...] * nf_ref[...]
         + jnp.dot(agg, na_ref[...], preferred_element_type=jnp.float32)
         + b1_ref[...])
    a = jnp.maximum(z, 0.0)
    y = jnp.dot(a, w2_ref[...], preferred_element_type=jnp.float32) + b2_ref[...]
    o_ref[...] = h + _ln_fused(y, g_ref[...], be_ref[...])


def _node_update(h_node, nfv, aggs1, aggs2, Nh, Nf, Na, b1, W2, b2, g, be):
    Nn, H = h_node.shape
    blk = pl.BlockSpec((BN, H), lambda i: (i, 0))
    return pl.pallas_call(
        _node_body,
        grid=(Nn // BN,),
        in_specs=[
            blk,
            pl.BlockSpec((BN, 1), lambda i: (i, 0)),
            blk, blk, blk, blk,
            pl.BlockSpec((H, H), lambda i: (0, 0)),
            pl.BlockSpec((1, H), lambda i: (0, 0)),
            pl.BlockSpec((H, H), lambda i: (0, 0)),
            pl.BlockSpec((H,), lambda i: (0,)),
            pl.BlockSpec((H, H), lambda i: (0, 0)),
            pl.BlockSpec((H,), lambda i: (0,)),
            pl.BlockSpec((H,), lambda i: (0,)),
            pl.BlockSpec((H,), lambda i: (0,)),
        ],
        out_specs=blk,
        out_shape=jax.ShapeDtypeStruct((Nn, H), jnp.float32),
    )(h_node, nfv, aggs1[0], aggs1[1], aggs2[0], aggs2[1],
      Nh, Nf, Na, b1, W2, b2, g, be)


def _dec_body(h_ref, nfv_ref, w1h_ref, w1f_ref, b1_ref, w2_ref, b2_ref, o_ref):
    z = (jnp.dot(h_ref[...], w1h_ref[...], preferred_element_type=jnp.float32)
         + nfv_ref[...] * w1f_ref[...] + b1_ref[...])
    a = jnp.maximum(z, 0.0)
    o_ref[...] = jnp.dot(a, w2_ref[...], preferred_element_type=jnp.float32) + b2_ref[...]


def _decode(h_node, nfv, W1h, W1f, b1, W2, b2):
    Nn, H = h_node.shape
    D1 = W1h.shape[1]
    DO = W2.shape[1]
    return pl.pallas_call(
        _dec_body,
        grid=(Nn // BN,),
        in_specs=[
            pl.BlockSpec((BN, H), lambda i: (i, 0)),
            pl.BlockSpec((BN, 1), lambda i: (i, 0)),
            pl.BlockSpec((H, D1), lambda i: (0, 0)),
            pl.BlockSpec((1, D1), lambda i: (0, 0)),
            pl.BlockSpec((D1,), lambda i: (0,)),
            pl.BlockSpec((D1, DO), lambda i: (0, 0)),
            pl.BlockSpec((DO,), lambda i: (0,)),
        ],
        out_specs=pl.BlockSpec((BN, DO), lambda i: (i, 0)),
        out_shape=jax.ShapeDtypeStruct((Nn, DO), jnp.float32),
    )(h_node, nfv, W1h, W1f, b1, W2, b2)


def _sc_gather(tableA, tableB, idxA, idxB):
    Erows = idxA.shape[0]
    Hd = tableA.shape[1]
    EW = Erows // NW
    K = KCH
    nchunk = EW // K
    npair = nchunk // 2
    mesh = plsc.VectorSubcoreMesh(core_axis_name="c", subcore_axis_name="s")

    def body(ta_hbm, tb_hbm, idxa_hbm, idxb_hbm, out_hbm,
             idxa_v, idxb_v, a0, b0, a1, b1, sa0, sb0, sa1, sb1):
        wid = lax.axis_index("s") * NC + lax.axis_index("c")
        base = wid * EW
        pltpu.sync_copy(idxa_hbm.at[pl.ds(base, EW)], idxa_v)
        pltpu.sync_copy(idxb_hbm.at[pl.ds(base, EW)], idxb_v)

        def addrows(dst, srcb):
            def row(r, carry):
                for j in range(Hd // 16):
                    s = (r, pl.ds(j * 16, 16))
                    dst[s] = dst[s] + srcb[s]
                return carry
            lax.fori_loop(0, K, row, 0)

        def chunk(c, bufa, bufb, sema, semb):
            ha = pltpu.async_copy(ta_hbm.at[idxa_v.at[pl.ds(c, K)]], bufa, sema)
            hb = pltpu.async_copy(tb_hbm.at[idxb_v.at[pl.ds(c, K)]], bufb, semb)
            return ha, hb

        def finish(c, bufa, bufb, ha, hb):
            ha.wait()
            hb.wait()
            addrows(bufa, bufb)
            pltpu.sync_copy(bufa, out_hbm.at[pl.ds(base + c, K)])

        def pair(g, carry):
            c0 = 2 * g * K
            c1 = c0 + K
            h0 = chunk(c0, a0, b0, sa0, sb0)
            h1 = chunk(c1, a1, b1, sa1, sb1)
            finish(c0, a0, b0, *h0)
            finish(c1, a1, b1, *h1)
            return carry

        lax.fori_loop(0, npair, pair, 0)
        if nchunk % 2 == 1:
            ct = (nchunk - 1) * K
            ht = chunk(ct, a0, b0, sa0, sb0)
            finish(ct, a0, b0, *ht)

    return pl.kernel(
        body,
        out_type=jax.ShapeDtypeStruct((Erows, Hd), jnp.float32),
        mesh=mesh,
        scratch_types=[
            pltpu.VMEM((EW,), jnp.int32),
            pltpu.VMEM((EW,), jnp.int32),
            pltpu.VMEM((K, Hd), jnp.float32),
            pltpu.VMEM((K, Hd), jnp.float32),
            pltpu.VMEM((K, Hd), jnp.float32),
            pltpu.VMEM((K, Hd), jnp.float32),
            pltpu.SemaphoreType.DMA,
            pltpu.SemaphoreType.DMA,
            pltpu.SemaphoreType.DMA,
            pltpu.SemaphoreType.DMA,
        ],
    )(tableA, tableB, idxA, idxB)


def _sc_scatter(rows, idx, zeros):
    Erows, Hd = rows.shape
    Nn = zeros.shape[0]
    EW = Erows // NW
    nchunk = EW // KSC
    mesh = plsc.VectorSubcoreMesh(core_axis_name="c", subcore_axis_name="s")

    def body(rows_hbm, idx_hbm, zeros_hbm, out_hbm, idx_v, rows_v, accum, sem):
        cid = lax.axis_index("c")
        sid = lax.axis_index("s")
        wid = sid * NC + cid

        @pl.when(sid == 0)
        def _():
            pltpu.sync_copy(zeros_hbm, accum)

        plsc.subcore_barrier()
        base = wid * EW

        def chunk(c, carry):
            off = base + c * KSC
            pltpu.sync_copy(idx_hbm.at[pl.ds(off, KSC)], idx_v)
            pltpu.async_copy(rows_hbm.at[pl.ds(off, KSC)], rows_v, sem).wait()
            pltpu.sync_copy(rows_v, accum.at[idx_v], add=True)
            return carry

        lax.fori_loop(0, nchunk, chunk, 0)
        plsc.subcore_barrier()

        @pl.when(sid == 0)
        def _():
            pltpu.sync_copy(accum, out_hbm.at[cid])

    return pl.kernel(
        body,
        out_type=jax.ShapeDtypeStruct((NC, Nn, Hd), jnp.float32),
        mesh=mesh,
        scratch_types=[
            pltpu.VMEM((KSC,), jnp.int32),
            pltpu.VMEM((KSC, Hd), jnp.float32),
            pltpu.VMEM_SHARED((Nn, Hd), jnp.float32),
            pltpu.SemaphoreType.DMA,
        ],
    )(rows, idx, zeros)


def kernel(x, edge_attr, edge_index, node_FVattr, edge_FVattr, params):
    H = 128
    Nn = x.shape[0]
    E = edge_index.shape[1]
    Eh = E // 2
    src1, src2 = edge_index[0, :Eh], edge_index[0, Eh:]
    dst1, dst2 = edge_index[1, :Eh], edge_index[1, Eh:]
    nfv = node_FVattr
    efv1, efv2 = edge_FVattr[:Eh], edge_FVattr[Eh:]
    zeros = jnp.zeros((Nn, H), jnp.float32)

    x_in = jnp.concatenate([x, nfv], axis=1)
    e_in = jnp.concatenate([edge_attr, edge_FVattr], axis=1)
    (We1, be1), (We2, be2) = params['enc_node_mlp']
    gn, bn = params['enc_node_ln']
    h_node = _encode(x_in, We1, be1, We2, be2, gn, bn, BN)
    (Wf1, bf1), (Wf2, bf2) = params['enc_edge_mlp']
    ge, bse = params['enc_edge_ln']
    h_edge1 = _encode(e_in[:Eh], Wf1, bf1, Wf2, bf2, ge, bse, BE)
    h_edge2 = _encode(e_in[Eh:], Wf1, bf1, Wf2, bf2, ge, bse, BE)

    def stk(f):
        return jnp.stack([f(c) for c in params['convs']])

    cw = {
        'Ah': stk(lambda c: c['edge_mlp'][0][0][0:H]),
        'Af': stk(lambda c: c['edge_mlp'][0][0][H:H + 1]),
        'Bh': stk(lambda c: c['edge_mlp'][0][0][H + 1:2 * H + 1]),
        'Bf': stk(lambda c: c['edge_mlp'][0][0][2 * H + 1:2 * H + 2]),
        'Ch': stk(lambda c: c['edge_mlp'][0][0][2 * H + 2:3 * H + 2]),
        'Cf': stk(lambda c: c['edge_mlp'][0][0][3 * H + 2:]),
        'eb1': stk(lambda c: c['edge_mlp'][0][1]),
        'eW2': stk(lambda c: c['edge_mlp'][1][0]),
        'eb2': stk(lambda c: c['edge_mlp'][1][1]),
        'eg': stk(lambda c: c['edge_ln'][0]),
        'ebeta': stk(lambda c: c['edge_ln'][1]),
        'Nh': stk(lambda c: c['node_mlp'][0][0][0:H]),
        'Nf': stk(lambda c: c['node_mlp'][0][0][H:H + 1]),
        'Na': stk(lambda c: c['node_mlp'][0][0][H + 1:]),
        'nb1': stk(lambda c: c['node_mlp'][0][1]),
        'nW2': stk(lambda c: c['node_mlp'][1][0]),
        'nb2': stk(lambda c: c['node_mlp'][1][1]),
        'ng': stk(lambda c: c['node_ln'][0]),
        'nbeta': stk(lambda c: c['node_ln'][1]),
    }

    def conv_step(carry, w):
        h_node, h_edge1, h_edge2 = carry
        PA, PB = _project(h_node, nfv, w['Ah'], w['Af'], w['Bh'], w['Bf'])
        G1 = _sc_gather(PA, PB, src1, dst1)
        G2 = _sc_gather(PA, PB, src2, dst2)
        h_edge1 = _edge_update(h_edge1, G1, efv1, w['Ch'], w['Cf'],
                               w['eb1'], w['eW2'], w['eb2'], w['eg'], w['ebeta'])
        aggs1 = _sc_scatter(h_edge1, dst1, zeros)
        h_edge2 = _edge_update(h_edge2, G2, efv2, w['Ch'], w['Cf'],
                               w['eb1'], w['eW2'], w['eb2'], w['eg'], w['ebeta'])
        aggs2 = _sc_scatter(h_edge2, dst2, zeros)
        h_node = _node_update(h_node, nfv, aggs1, aggs2, w['Nh'], w['Nf'],
                              w['Na'], w['nb1'], w['nW2'], w['nb2'],
                              w['ng'], w['nbeta'])
        return (h_node, h_edge1, h_edge2), None

    (h_node, h_edge1, h_edge2), _ = lax.scan(
        conv_step, (h_node, h_edge1, h_edge2), cw)

    (Wd1, bd1), (Wd2, bd2) = params['dec_mlp']
    return _decode(h_node, nfv, Wd1[0:H], Wd1[H:H + 1], bd1, Wd2, bd2)


# speedup vs baseline: 3.9856x; 1.0079x over previous
import functools
import jax
import jax.numpy as jnp
import numpy as np
from jax import lax
from jax.experimental import pallas as pl
from jax.experimental.pallas import tpu as pltpu
from jax.experimental.pallas import tpu_sc as plsc

NC = 2
NS = 16
NW = NC * NS

BE = 2000
BN = 2000
KCH = 200
KSC = 192


def _ln_fused(y, g, b):
    m = jnp.mean(y, axis=-1, keepdims=True)
    d = y - m
    v = jnp.mean(d * d, axis=-1, keepdims=True)
    return d * lax.rsqrt(v + 1e-5) * g + b


def _enc_body(x_ref, w1_ref, b1_ref, w2_ref, b2_ref, g_ref, be_ref, o_ref):
    a = jnp.maximum(
        jnp.dot(x_ref[...], w1_ref[...], preferred_element_type=jnp.float32)
        + b1_ref[...], 0.0)
    y = jnp.dot(a, w2_ref[...], preferred_element_type=jnp.float32) + b2_ref[...]
    o_ref[...] = _ln_fused(y, g_ref[...], be_ref[...])


def _encode(xin, W1, b1, W2, b2, g, be, BR):
    R, Din = xin.shape
    H = W2.shape[1]
    return pl.pallas_call(
        _enc_body,
        grid=(R // BR,),
        in_specs=[
            pl.BlockSpec((BR, Din), lambda i: (i, 0)),
            pl.BlockSpec((Din, H), lambda i: (0, 0)),
            pl.BlockSpec((H,), lambda i: (0,)),
            pl.BlockSpec((H, H), lambda i: (0, 0)),
            pl.BlockSpec((H,), lambda i: (0,)),
            pl.BlockSpec((H,), lambda i: (0,)),
            pl.BlockSpec((H,), lambda i: (0,)),
        ],
        out_specs=pl.BlockSpec((BR, H), lambda i: (i, 0)),
        out_shape=jax.ShapeDtypeStruct((R, H), jnp.float32),
    )(xin, W1, b1, W2, b2, g, be)


def _proj_body(h_ref, nfv_ref, ah_ref, af_ref, bh_ref, bf_ref, pa_ref, pb_ref):
    h = h_ref[...]
    nfv = nfv_ref[...]
    pa_ref[...] = jnp.dot(h, ah_ref[...], preferred_element_type=jnp.float32) + nfv * af_ref[...]
    pb_ref[...] = jnp.dot(h, bh_ref[...], preferred_element_type=jnp.float32) + nfv * bf_ref[...]


def _project(h_node, nfv, Ah, Af, Bh, Bf):
    Nn, H = h_node.shape
    out = jax.ShapeDtypeStruct((Nn, H), jnp.float32)
    return pl.pallas_call(
        _proj_body,
        grid=(Nn // BN,),
        in_specs=[
            pl.BlockSpec((BN, H), lambda i: (i, 0)),
            pl.BlockSpec((BN, 1), lambda i: (i, 0)),
            pl.BlockSpec((H, H), lambda i: (0, 0)),
            pl.BlockSpec((1, H), lambda i: (0, 0)),
            pl.BlockSpec((H, H), lambda i: (0, 0)),
            pl.BlockSpec((1, H), lambda i: (0, 0)),
        ],
        out_specs=[
            pl.BlockSpec((BN, H), lambda i: (i, 0)),
            pl.BlockSpec((BN, H), lambda i: (i, 0)),
        ],
        out_shape=[out, out],
    )(h_node, nfv, Ah, Af, Bh, Bf)


def _edge_body(he_ref, ga_ref, efv_ref, ch_ref, cf_ref, b1_ref,
               w2_ref, b2_ref, g_ref, be_ref, o_ref):
    he = he_ref[...]
    z = (ga_ref[...]
         + jnp.dot(he, ch_ref[...], preferred_element_type=jnp.float32)
         + jnp.dot(efv_ref[...], cf_ref[...], preferred_element_type=jnp.float32)
         + b1_ref[...])
    a = jnp.maximum(z, 0.0)
    y = jnp.dot(a, w2_ref[...], preferred_element_type=jnp.float32) + b2_ref[...]
    o_ref[...] = he + _ln_fused(y, g_ref[...], be_ref[...])


def _edge_update(h_edge, G, efv, Ch, Cf, b1, W2, b2, g, be):
    E, H = h_edge.shape
    F = efv.shape[1]
    return pl.pallas_call(
        _edge_body,
        grid=(E // BE,),
        in_specs=[
            pl.BlockSpec((BE, H), lambda i: (i, 0)),
            pl.BlockSpec((BE, H), lambda i: (i, 0)),
            pl.BlockSpec((BE, F), lambda i: (i, 0)),
            pl.BlockSpec((H, H), lambda i: (0, 0)),
            pl.BlockSpec((F, H), lambda i: (0, 0)),
            pl.BlockSpec((H,), lambda i: (0,)),
            pl.BlockSpec((H, H), lambda i: (0, 0)),
            pl.BlockSpec((H,), lambda i: (0,)),
            pl.BlockSpec((H,), lambda i: (0,)),
            pl.BlockSpec((H,), lambda i: (0,)),
        ],
        out_specs=pl.BlockSpec((BE, H), lambda i: (i, 0)),
        out_shape=jax.ShapeDtypeStruct((E, H), jnp.float32),
    )(h_edge, G, efv, Ch, Cf, b1, W2, b2, g, be)


def _node_body(h_ref, nfv_ref, a00_ref, a01_ref, a10_ref, a11_ref,
               nh_ref, nf_ref, na_ref,
               b1_ref, w2_ref, b2_ref, g_ref, be_ref, o_ref):
    h = h_ref[...]
    agg = ((a00_ref[...] + a01_ref[...]) + (a10_ref[...] + a11_ref[...]))
    z = (jnp.dot(h, nh_ref[...], preferred_element_type=jnp.float32)
         + nfv_ref[...] * nf_ref[...]
         + jnp.dot(agg, na_ref[...], preferred_element_type=jnp.float32)
         + b1_ref[...])
    a = jnp.maximum(z, 0.0)
    y = jnp.dot(a, w2_ref[...], preferred_element_type=jnp.float32) + b2_ref[...]
    o_ref[...] = h + _ln_fused(y, g_ref[...], be_ref[...])


def _node_update(h_node, nfv, aggs1, aggs2, Nh, Nf, Na, b1, W2, b2, g, be):
    Nn, H = h_node.shape
    blk = pl.BlockSpec((BN, H), lambda i: (i, 0))
    return pl.pallas_call(
        _node_body,
        grid=(Nn // BN,),
        in_specs=[
            blk,
            pl.BlockSpec((BN, 1), lambda i: (i, 0)),
            blk, blk, blk, blk,
            pl.BlockSpec((H, H), lambda i: (0, 0)),
            pl.BlockSpec((1, H), lambda i: (0, 0)),
            pl.BlockSpec((H, H), lambda i: (0, 0)),
            pl.BlockSpec((H,), lambda i: (0,)),
            pl.BlockSpec((H, H), lambda i: (0, 0)),
            pl.BlockSpec((H,), lambda i: (0,)),
            pl.BlockSpec((H,), lambda i: (0,)),
            pl.BlockSpec((H,), lambda i: (0,)),
        ],
        out_specs=blk,
        out_shape=jax.ShapeDtypeStruct((Nn, H), jnp.float32),
    )(h_node, nfv, aggs1[0], aggs1[1], aggs2[0], aggs2[1],
      Nh, Nf, Na, b1, W2, b2, g, be)


def _dec_body(h_ref, nfv_ref, w1h_ref, w1f_ref, b1_ref, w2_ref, b2_ref, o_ref):
    z = (jnp.dot(h_ref[...], w1h_ref[...], preferred_element_type=jnp.float32)
         + nfv_ref[...] * w1f_ref[...] + b1_ref[...])
    a = jnp.maximum(z, 0.0)
    o_ref[...] = jnp.dot(a, w2_ref[...], preferred_element_type=jnp.float32) + b2_ref[...]


def _decode(h_node, nfv, W1h, W1f, b1, W2, b2):
    Nn, H = h_node.shape
    D1 = W1h.shape[1]
    DO = W2.shape[1]
    return pl.pallas_call(
        _dec_body,
        grid=(Nn // BN,),
        in_specs=[
            pl.BlockSpec((BN, H), lambda i: (i, 0)),
            pl.BlockSpec((BN, 1), lambda i: (i, 0)),
            pl.BlockSpec((H, D1), lambda i: (0, 0)),
            pl.BlockSpec((1, D1), lambda i: (0, 0)),
            pl.BlockSpec((D1,), lambda i: (0,)),
            pl.BlockSpec((D1, DO), lambda i: (0, 0)),
            pl.BlockSpec((DO,), lambda i: (0,)),
        ],
        out_specs=pl.BlockSpec((BN, DO), lambda i: (i, 0)),
        out_shape=jax.ShapeDtypeStruct((Nn, DO), jnp.float32),
    )(h_node, nfv, W1h, W1f, b1, W2, b2)


def _sc_gather(tableA, tableB, idxA, idxB):
    Erows = idxA.shape[0]
    Hd = tableA.shape[1]
    EW = Erows // NW
    K = KCH
    nchunk = EW // K
    npair = nchunk // 2
    mesh = plsc.VectorSubcoreMesh(core_axis_name="c", subcore_axis_name="s")

    def body(ta_hbm, tb_hbm, idxa_hbm, idxb_hbm, out_hbm,
             idxa_v, idxb_v, a0, b0, a1, b1, sa0, sb0, sa1, sb1):
        wid = lax.axis_index("s") * NC + lax.axis_index("c")
        base = wid * EW
        pltpu.sync_copy(idxa_hbm.at[pl.ds(base, EW)], idxa_v)
        pltpu.sync_copy(idxb_hbm.at[pl.ds(base, EW)], idxb_v)

        def addrows(dst, srcb):
            def row(r, carry):
                for j in range(Hd // 16):
                    s = (r, pl.ds(j * 16, 16))
                    dst[s] = dst[s] + srcb[s]
                return carry
            lax.fori_loop(0, K, row, 0)

        def chunk(c, bufa, bufb, sema, semb):
            ha = pltpu.async_copy(ta_hbm.at[idxa_v.at[pl.ds(c, K)]], bufa, sema)
            hb = pltpu.async_copy(tb_hbm.at[idxb_v.at[pl.ds(c, K)]], bufb, semb)
            return ha, hb

        def finish(c, bufa, bufb, ha, hb):
            ha.wait()
            hb.wait()
            addrows(bufa, bufb)
            pltpu.sync_copy(bufa, out_hbm.at[pl.ds(base + c, K)])

        def pair(g, carry):
            c0 = 2 * g * K
            c1 = c0 + K
            h0 = chunk(c0, a0, b0, sa0, sb0)
            h1 = chunk(c1, a1, b1, sa1, sb1)
            finish(c0, a0, b0, *h0)
            finish(c1, a1, b1, *h1)
            return carry

        lax.fori_loop(0, npair, pair, 0)
        if nchunk % 2 == 1:
            ct = (nchunk - 1) * K
            ht = chunk(ct, a0, b0, sa0, sb0)
            finish(ct, a0, b0, *ht)

    return pl.kernel(
        body,
        out_type=jax.ShapeDtypeStruct((Erows, Hd), jnp.float32),
        mesh=mesh,
        scratch_types=[
            pltpu.VMEM((EW,), jnp.int32),
            pltpu.VMEM((EW,), jnp.int32),
            pltpu.VMEM((K, Hd), jnp.float32),
            pltpu.VMEM((K, Hd), jnp.float32),
            pltpu.VMEM((K, Hd), jnp.float32),
            pltpu.VMEM((K, Hd), jnp.float32),
            pltpu.SemaphoreType.DMA,
            pltpu.SemaphoreType.DMA,
            pltpu.SemaphoreType.DMA,
            pltpu.SemaphoreType.DMA,
        ],
    )(tableA, tableB, idxA, idxB)


def _sc_scatter(rows, idx, zeros):
    Erows, Hd = rows.shape
    Nn = zeros.shape[0]
    EW = Erows // NW
    K = KSC
    nfull = EW // K
    tail = EW - nfull * K
    npair = nfull // 2
    mesh = plsc.VectorSubcoreMesh(core_axis_name="c", subcore_axis_name="s")

    def body(rows_hbm, idx_hbm, zeros_hbm, out_hbm,
             i0, i1, it, r0, r1, rt, accum, s0, s1):
        cid = lax.axis_index("c")
        sid = lax.axis_index("s")
        wid = sid * NC + cid

        @pl.when(sid == 0)
        def _():
            pltpu.sync_copy(zeros_hbm, accum)

        plsc.subcore_barrier()
        base = wid * EW

        def start(c, ibuf, rbuf, sem, n):
            pltpu.sync_copy(idx_hbm.at[pl.ds(base + c, n)], ibuf)
            return pltpu.async_copy(rows_hbm.at[pl.ds(base + c, n)], rbuf, sem)

        def finish(h, ibuf, rbuf):
            h.wait()
            pltpu.sync_copy(rbuf, accum.at[ibuf], add=True)

        def pair(g, carry):
            c0 = 2 * g * K
            c1 = c0 + K
            h0 = start(c0, i0, r0, s0, K)
            h1 = start(c1, i1, r1, s1, K)
            finish(h0, i0, r0)
            finish(h1, i1, r1)
            return carry

        lax.fori_loop(0, npair, pair, 0)
        if nfull % 2 == 1:
            co = (nfull - 1) * K
            ho = start(co, i0, r0, s0, K)
            finish(ho, i0, r0)
        if tail:
            ht = start(nfull * K, it, rt, s1, tail)
            finish(ht, it, rt)
        plsc.subcore_barrier()

        @pl.when(sid == 0)
        def _():
            pltpu.sync_copy(accum, out_hbm.at[cid])

    scratch = [
        pltpu.VMEM((K,), jnp.int32),
        pltpu.VMEM((K,), jnp.int32),
        pltpu.VMEM((max(tail, 8),), jnp.int32),
        pltpu.VMEM((K, Hd), jnp.float32),
        pltpu.VMEM((K, Hd), jnp.float32),
        pltpu.VMEM((max(tail, 8), Hd), jnp.float32),
        pltpu.VMEM_SHARED((Nn, Hd), jnp.float32),
        pltpu.SemaphoreType.DMA,
        pltpu.SemaphoreType.DMA,
    ]
    return pl.kernel(
        body,
        out_type=jax.ShapeDtypeStruct((NC, Nn, Hd), jnp.float32),
        mesh=mesh,
        scratch_types=scratch,
    )(rows, idx, zeros)


def kernel(x, edge_attr, edge_index, node_FVattr, edge_FVattr, params):
    H = 128
    Nn = x.shape[0]
    E = edge_index.shape[1]
    Eh = E // 2
    src1, src2 = edge_index[0, :Eh], edge_index[0, Eh:]
    dst1, dst2 = edge_index[1, :Eh], edge_index[1, Eh:]
    nfv = node_FVattr
    efv1, efv2 = edge_FVattr[:Eh], edge_FVattr[Eh:]
    zeros = jnp.zeros((Nn, H), jnp.float32)

    x_in = jnp.concatenate([x, nfv], axis=1)
    e_in = jnp.concatenate([edge_attr, edge_FVattr], axis=1)
    (We1, be1), (We2, be2) = params['enc_node_mlp']
    gn, bn = params['enc_node_ln']
    h_node = _encode(x_in, We1, be1, We2, be2, gn, bn, BN)
    (Wf1, bf1), (Wf2, bf2) = params['enc_edge_mlp']
    ge, bse = params['enc_edge_ln']
    h_edge1 = _encode(e_in[:Eh], Wf1, bf1, Wf2, bf2, ge, bse, BE)
    h_edge2 = _encode(e_in[Eh:], Wf1, bf1, Wf2, bf2, ge, bse, BE)

    def stk(f):
        return jnp.stack([f(c) for c in params['convs']])

    cw = {
        'Ah': stk(lambda c: c['edge_mlp'][0][0][0:H]),
        'Af': stk(lambda c: c['edge_mlp'][0][0][H:H + 1]),
        'Bh': stk(lambda c: c['edge_mlp'][0][0][H + 1:2 * H + 1]),
        'Bf': stk(lambda c: c['edge_mlp'][0][0][2 * H + 1:2 * H + 2]),
        'Ch': stk(lambda c: c['edge_mlp'][0][0][2 * H + 2:3 * H + 2]),
        'Cf': stk(lambda c: c['edge_mlp'][0][0][3 * H + 2:]),
        'eb1': stk(lambda c: c['edge_mlp'][0][1]),
        'eW2': stk(lambda c: c['edge_mlp'][1][0]),
        'eb2': stk(lambda c: c['edge_mlp'][1][1]),
        'eg': stk(lambda c: c['edge_ln'][0]),
        'ebeta': stk(lambda c: c['edge_ln'][1]),
        'Nh': stk(lambda c: c['node_mlp'][0][0][0:H]),
        'Nf': stk(lambda c: c['node_mlp'][0][0][H:H + 1]),
        'Na': stk(lambda c: c['node_mlp'][0][0][H + 1:]),
        'nb1': stk(lambda c: c['node_mlp'][0][1]),
        'nW2': stk(lambda c: c['node_mlp'][1][0]),
        'nb2': stk(lambda c: c['node_mlp'][1][1]),
        'ng': stk(lambda c: c['node_ln'][0]),
        'nbeta': stk(lambda c: c['node_ln'][1]),
    }

    def conv_step(carry, w):
        h_node, h_edge1, h_edge2 = carry
        PA, PB = _project(h_node, nfv, w['Ah'], w['Af'], w['Bh'], w['Bf'])
        G1 = _sc_gather(PA, PB, src1, dst1)
        G2 = _sc_gather(PA, PB, src2, dst2)
        h_edge1 = _edge_update(h_edge1, G1, efv1, w['Ch'], w['Cf'],
                               w['eb1'], w['eW2'], w['eb2'], w['eg'], w['ebeta'])
        aggs1 = _sc_scatter(h_edge1, dst1, zeros)
        h_edge2 = _edge_update(h_edge2, G2, efv2, w['Ch'], w['Cf'],
                               w['eb1'], w['eW2'], w['eb2'], w['eg'], w['ebeta'])
        aggs2 = _sc_scatter(h_edge2, dst2, zeros)
        h_node = _node_update(h_node, nfv, aggs1, aggs2, w['Nh'], w['Nf'],
                              w['Na'], w['nb1'], w['nW2'], w['nb2'],
                              w['ng'], w['nbeta'])
        return (h_node, h_edge1, h_edge2), None

    (h_node, h_edge1, h_edge2), _ = lax.scan(
        conv_step, (h_node, h_edge1, h_edge2), cw)

    (Wd1, bd1), (Wd2, bd2) = params['dec_mlp']
    return _decode(h_node, nfv, Wd1[0:H], Wd1[H:H + 1], bd1, Wd2, bd2)


# speedup vs baseline: 4.0575x; 1.0180x over previous
import functools
import jax
import jax.numpy as jnp
import numpy as np
from jax import lax
from jax.experimental import pallas as pl
from jax.experimental.pallas import tpu as pltpu
from jax.experimental.pallas import tpu_sc as plsc

NC = 2
NS = 16
NW = NC * NS

BE = 2000
BN = 2000
KCH = 200
KSC = 192


def _ln_fused(y, g, b):
    m = jnp.mean(y, axis=-1, keepdims=True)
    d = y - m
    v = jnp.mean(d * d, axis=-1, keepdims=True)
    return d * lax.rsqrt(v + 1e-5) * g + b


def _enc_body(x_ref, w1_ref, b1_ref, w2_ref, b2_ref, g_ref, be_ref, o_ref):
    a = jnp.maximum(
        jnp.dot(x_ref[...], w1_ref[...], preferred_element_type=jnp.float32)
        + b1_ref[...], 0.0)
    y = jnp.dot(a, w2_ref[...], preferred_element_type=jnp.float32) + b2_ref[...]
    o_ref[...] = _ln_fused(y, g_ref[...], be_ref[...])


def _encode(xin, W1, b1, W2, b2, g, be, BR):
    R, Din = xin.shape
    H = W2.shape[1]
    return pl.pallas_call(
        _enc_body,
        grid=(R // BR,),
        in_specs=[
            pl.BlockSpec((BR, Din), lambda i: (i, 0)),
            pl.BlockSpec((Din, H), lambda i: (0, 0)),
            pl.BlockSpec((H,), lambda i: (0,)),
            pl.BlockSpec((H, H), lambda i: (0, 0)),
            pl.BlockSpec((H,), lambda i: (0,)),
            pl.BlockSpec((H,), lambda i: (0,)),
            pl.BlockSpec((H,), lambda i: (0,)),
        ],
        out_specs=pl.BlockSpec((BR, H), lambda i: (i, 0)),
        out_shape=jax.ShapeDtypeStruct((R, H), jnp.float32),
    )(xin, W1, b1, W2, b2, g, be)


def _proj_body(h_ref, nfv_ref, ah_ref, af_ref, bh_ref, bf_ref, pa_ref, pb_ref):
    h = h_ref[...]
    nfv = nfv_ref[...]
    pa_ref[...] = jnp.dot(h, ah_ref[...], preferred_element_type=jnp.float32) + nfv * af_ref[...]
    pb_ref[...] = jnp.dot(h, bh_ref[...], preferred_element_type=jnp.float32) + nfv * bf_ref[...]


def _project(h_node, nfv, Ah, Af, Bh, Bf):
    Nn, H = h_node.shape
    out = jax.ShapeDtypeStruct((Nn, H), jnp.float32)
    return pl.pallas_call(
        _proj_body,
        grid=(Nn // BN,),
        in_specs=[
            pl.BlockSpec((BN, H), lambda i: (i, 0)),
            pl.BlockSpec((BN, 1), lambda i: (i, 0)),
            pl.BlockSpec((H, H), lambda i: (0, 0)),
            pl.BlockSpec((1, H), lambda i: (0, 0)),
            pl.BlockSpec((H, H), lambda i: (0, 0)),
            pl.BlockSpec((1, H), lambda i: (0, 0)),
        ],
        out_specs=[
            pl.BlockSpec((BN, H), lambda i: (i, 0)),
            pl.BlockSpec((BN, H), lambda i: (i, 0)),
        ],
        out_shape=[out, out],
    )(h_node, nfv, Ah, Af, Bh, Bf)


def _edge_body(he_ref, ga_ref, efv_ref, ch_ref, cf_ref, b1_ref,
               w2_ref, b2_ref, g_ref, be_ref, o_ref):
    he = he_ref[...]
    z = (ga_ref[...]
         + jnp.dot(he, ch_ref[...], preferred_element_type=jnp.float32)
         + jnp.dot(efv_ref[...], cf_ref[...], preferred_element_type=jnp.float32)
         + b1_ref[...])
    a = jnp.maximum(z, 0.0)
    y = jnp.dot(a, w2_ref[...], preferred_element_type=jnp.float32) + b2_ref[...]
    o_ref[...] = he + _ln_fused(y, g_ref[...], be_ref[...])


def _edge_update(h_edge, G, efv, Ch, Cf, b1, W2, b2, g, be):
    E, H = h_edge.shape
    F = efv.shape[1]
    return pl.pallas_call(
        _edge_body,
        grid=(E // BE,),
        in_specs=[
            pl.BlockSpec((BE, H), lambda i: (i, 0)),
            pl.BlockSpec((BE, H), lambda i: (i, 0)),
            pl.BlockSpec((BE, F), lambda i: (i, 0)),
            pl.BlockSpec((H, H), lambda i: (0, 0)),
            pl.BlockSpec((F, H), lambda i: (0, 0)),
            pl.BlockSpec((H,), lambda i: (0,)),
            pl.BlockSpec((H, H), lambda i: (0, 0)),
            pl.BlockSpec((H,), lambda i: (0,)),
            pl.BlockSpec((H,), lambda i: (0,)),
            pl.BlockSpec((H,), lambda i: (0,)),
        ],
        out_specs=pl.BlockSpec((BE, H), lambda i: (i, 0)),
        out_shape=jax.ShapeDtypeStruct((E, H), jnp.float32),
    )(h_edge, G, efv, Ch, Cf, b1, W2, b2, g, be)


def _node_body(h_ref, nfv_ref, a00_ref, a01_ref, a10_ref, a11_ref,
               nh_ref, nf_ref, na_ref,
               b1_ref, w2_ref, b2_ref, g_ref, be_ref,
               ah_ref, af_ref, bh_ref, bf_ref,
               o_ref, pa_ref, pb_ref):
    h = h_ref[...]
    nfv = nfv_ref[...]
    agg = ((a00_ref[...] + a01_ref[...]) + (a10_ref[...] + a11_ref[...]))
    z = (jnp.dot(h, nh_ref[...], preferred_element_type=jnp.float32)
         + nfv * nf_ref[...]
         + jnp.dot(agg, na_ref[...], preferred_element_type=jnp.float32)
         + b1_ref[...])
    a = jnp.maximum(z, 0.0)
    y = jnp.dot(a, w2_ref[...], preferred_element_type=jnp.float32) + b2_ref[...]
    hn = h + _ln_fused(y, g_ref[...], be_ref[...])
    o_ref[...] = hn
    pa_ref[...] = jnp.dot(hn, ah_ref[...], preferred_element_type=jnp.float32) + nfv * af_ref[...]
    pb_ref[...] = jnp.dot(hn, bh_ref[...], preferred_element_type=jnp.float32) + nfv * bf_ref[...]


def _node_update(h_node, nfv, aggs1, aggs2, Nh, Nf, Na, b1, W2, b2, g, be,
                 Ah2, Af2, Bh2, Bf2):
    Nn, H = h_node.shape
    blk = pl.BlockSpec((BN, H), lambda i: (i, 0))
    wblk = pl.BlockSpec((H, H), lambda i: (0, 0))
    rblk = pl.BlockSpec((1, H), lambda i: (0, 0))
    vblk = pl.BlockSpec((H,), lambda i: (0,))
    out = jax.ShapeDtypeStruct((Nn, H), jnp.float32)
    return pl.pallas_call(
        _node_body,
        grid=(Nn // BN,),
        in_specs=[
            blk,
            pl.BlockSpec((BN, 1), lambda i: (i, 0)),
            blk, blk, blk, blk,
            wblk, rblk, wblk, vblk, wblk, vblk, vblk, vblk,
            wblk, rblk, wblk, rblk,
        ],
        out_specs=[blk, blk, blk],
        out_shape=[out, out, out],
    )(h_node, nfv, aggs1[0], aggs1[1], aggs2[0], aggs2[1],
      Nh, Nf, Na, b1, W2, b2, g, be, Ah2, Af2, Bh2, Bf2)


def _dec_body(h_ref, nfv_ref, w1h_ref, w1f_ref, b1_ref, w2_ref, b2_ref, o_ref):
    z = (jnp.dot(h_ref[...], w1h_ref[...], preferred_element_type=jnp.float32)
         + nfv_ref[...] * w1f_ref[...] + b1_ref[...])
    a = jnp.maximum(z, 0.0)
    o_ref[...] = jnp.dot(a, w2_ref[...], preferred_element_type=jnp.float32) + b2_ref[...]


def _decode(h_node, nfv, W1h, W1f, b1, W2, b2):
    Nn, H = h_node.shape
    D1 = W1h.shape[1]
    DO = W2.shape[1]
    return pl.pallas_call(
        _dec_body,
        grid=(Nn // BN,),
        in_specs=[
            pl.BlockSpec((BN, H), lambda i: (i, 0)),
            pl.BlockSpec((BN, 1), lambda i: (i, 0)),
            pl.BlockSpec((H, D1), lambda i: (0, 0)),
            pl.BlockSpec((1, D1), lambda i: (0, 0)),
            pl.BlockSpec((D1,), lambda i: (0,)),
            pl.BlockSpec((D1, DO), lambda i: (0, 0)),
            pl.BlockSpec((DO,), lambda i: (0,)),
        ],
        out_specs=pl.BlockSpec((BN, DO), lambda i: (i, 0)),
        out_shape=jax.ShapeDtypeStruct((Nn, DO), jnp.float32),
    )(h_node, nfv, W1h, W1f, b1, W2, b2)


def _sc_gather(tableA, tableB, idxA, idxB):
    Erows = idxA.shape[0]
    Hd = tableA.shape[1]
    EW = Erows // NW
    K = KCH
    nchunk = EW // K
    npair = nchunk // 2
    mesh = plsc.VectorSubcoreMesh(core_axis_name="c", subcore_axis_name="s")

    def body(ta_hbm, tb_hbm, idxa_hbm, idxb_hbm, out_hbm,
             idxa_v, idxb_v, a0, b0, a1, b1, sa0, sb0, sa1, sb1):
        wid = lax.axis_index("s") * NC + lax.axis_index("c")
        base = wid * EW
        pltpu.sync_copy(idxa_hbm.at[pl.ds(base, EW)], idxa_v)
        pltpu.sync_copy(idxb_hbm.at[pl.ds(base, EW)], idxb_v)

        def addrows(dst, srcb):
            def row(r, carry):
                for j in range(Hd // 16):
                    s = (r, pl.ds(j * 16, 16))
                    dst[s] = dst[s] + srcb[s]
                return carry
            lax.fori_loop(0, K, row, 0)

        def chunk(c, bufa, bufb, sema, semb):
            ha = pltpu.async_copy(ta_hbm.at[idxa_v.at[pl.ds(c, K)]], bufa, sema)
            hb = pltpu.async_copy(tb_hbm.at[idxb_v.at[pl.ds(c, K)]], bufb, semb)
            return ha, hb

        def finish(c, bufa, bufb, ha, hb):
            ha.wait()
            hb.wait()
            addrows(bufa, bufb)
            pltpu.sync_copy(bufa, out_hbm.at[pl.ds(base + c, K)])

        def pair(g, carry):
            c0 = 2 * g * K
            c1 = c0 + K
            h0 = chunk(c0, a0, b0, sa0, sb0)
            h1 = chunk(c1, a1, b1, sa1, sb1)
            finish(c0, a0, b0, *h0)
            finish(c1, a1, b1, *h1)
            return carry

        lax.fori_loop(0, npair, pair, 0)
        if nchunk % 2 == 1:
            ct = (nchunk - 1) * K
            ht = chunk(ct, a0, b0, sa0, sb0)
            finish(ct, a0, b0, *ht)

    return pl.kernel(
        body,
        out_type=jax.ShapeDtypeStruct((Erows, Hd), jnp.float32),
        mesh=mesh,
        scratch_types=[
            pltpu.VMEM((EW,), jnp.int32),
            pltpu.VMEM((EW,), jnp.int32),
            pltpu.VMEM((K, Hd), jnp.float32),
            pltpu.VMEM((K, Hd), jnp.float32),
            pltpu.VMEM((K, Hd), jnp.float32),
            pltpu.VMEM((K, Hd), jnp.float32),
            pltpu.SemaphoreType.DMA,
            pltpu.SemaphoreType.DMA,
            pltpu.SemaphoreType.DMA,
            pltpu.SemaphoreType.DMA,
        ],
    )(tableA, tableB, idxA, idxB)


def _sc_scatter(rows, idx, zeros):
    Erows, Hd = rows.shape
    Nn = zeros.shape[0]
    EW = Erows // NW
    K = KSC
    nfull = EW // K
    tail = EW - nfull * K
    npair = nfull // 2
    mesh = plsc.VectorSubcoreMesh(core_axis_name="c", subcore_axis_name="s")

    def body(rows_hbm, idx_hbm, zeros_hbm, out_hbm,
             i0, i1, it, r0, r1, rt, accum, s0, s1):
        cid = lax.axis_index("c")
        sid = lax.axis_index("s")
        wid = sid * NC + cid

        @pl.when(sid == 0)
        def _():
            pltpu.sync_copy(zeros_hbm, accum)

        plsc.subcore_barrier()
        base = wid * EW

        def start(c, ibuf, rbuf, sem, n):
            pltpu.sync_copy(idx_hbm.at[pl.ds(base + c, n)], ibuf)
            return pltpu.async_copy(rows_hbm.at[pl.ds(base + c, n)], rbuf, sem)

        def finish(h, ibuf, rbuf):
            h.wait()
            pltpu.sync_copy(rbuf, accum.at[ibuf], add=True)

        def pair(g, carry):
            c0 = 2 * g * K
            c1 = c0 + K
            h0 = start(c0, i0, r0, s0, K)
            h1 = start(c1, i1, r1, s1, K)
            finish(h0, i0, r0)
            finish(h1, i1, r1)
            return carry

        lax.fori_loop(0, npair, pair, 0)
        if nfull % 2 == 1:
            co = (nfull - 1) * K
            ho = start(co, i0, r0, s0, K)
            finish(ho, i0, r0)
        if tail:
            ht = start(nfull * K, it, rt, s1, tail)
            finish(ht, it, rt)
        plsc.subcore_barrier()

        @pl.when(sid == 0)
        def _():
            pltpu.sync_copy(accum, out_hbm.at[cid])

    scratch = [
        pltpu.VMEM((K,), jnp.int32),
        pltpu.VMEM((K,), jnp.int32),
        pltpu.VMEM((max(tail, 8),), jnp.int32),
        pltpu.VMEM((K, Hd), jnp.float32),
        pltpu.VMEM((K, Hd), jnp.float32),
        pltpu.VMEM((max(tail, 8), Hd), jnp.float32),
        pltpu.VMEM_SHARED((Nn, Hd), jnp.float32),
        pltpu.SemaphoreType.DMA,
        pltpu.SemaphoreType.DMA,
    ]
    return pl.kernel(
        body,
        out_type=jax.ShapeDtypeStruct((NC, Nn, Hd), jnp.float32),
        mesh=mesh,
        scratch_types=scratch,
    )(rows, idx, zeros)


def kernel(x, edge_attr, edge_index, node_FVattr, edge_FVattr, params):
    H = 128
    Nn = x.shape[0]
    E = edge_index.shape[1]
    Eh = E // 2
    src1, src2 = edge_index[0, :Eh], edge_index[0, Eh:]
    dst1, dst2 = edge_index[1, :Eh], edge_index[1, Eh:]
    nfv = node_FVattr
    efv1, efv2 = edge_FVattr[:Eh], edge_FVattr[Eh:]
    zeros = jnp.zeros((Nn, H), jnp.float32)

    x_in = jnp.concatenate([x, nfv], axis=1)
    e_in = jnp.concatenate([edge_attr, edge_FVattr], axis=1)
    (We1, be1), (We2, be2) = params['enc_node_mlp']
    gn, bn = params['enc_node_ln']
    h_node = _encode(x_in, We1, be1, We2, be2, gn, bn, BN)
    (Wf1, bf1), (Wf2, bf2) = params['enc_edge_mlp']
    ge, bse = params['enc_edge_ln']
    h_edge1 = _encode(e_in[:Eh], Wf1, bf1, Wf2, bf2, ge, bse, BE)
    h_edge2 = _encode(e_in[Eh:], Wf1, bf1, Wf2, bf2, ge, bse, BE)

    def stk(f):
        return jnp.stack([f(c) for c in params['convs']])

    cw = {
        'Ah': stk(lambda c: c['edge_mlp'][0][0][0:H]),
        'Af': stk(lambda c: c['edge_mlp'][0][0][H:H + 1]),
        'Bh': stk(lambda c: c['edge_mlp'][0][0][H + 1:2 * H + 1]),
        'Bf': stk(lambda c: c['edge_mlp'][0][0][2 * H + 1:2 * H + 2]),
        'Ch': stk(lambda c: c['edge_mlp'][0][0][2 * H + 2:3 * H + 2]),
        'Cf': stk(lambda c: c['edge_mlp'][0][0][3 * H + 2:]),
        'eb1': stk(lambda c: c['edge_mlp'][0][1]),
        'eW2': stk(lambda c: c['edge_mlp'][1][0]),
        'eb2': stk(lambda c: c['edge_mlp'][1][1]),
        'eg': stk(lambda c: c['edge_ln'][0]),
        'ebeta': stk(lambda c: c['edge_ln'][1]),
        'Nh': stk(lambda c: c['node_mlp'][0][0][0:H]),
        'Nf': stk(lambda c: c['node_mlp'][0][0][H:H + 1]),
        'Na': stk(lambda c: c['node_mlp'][0][0][H + 1:]),
        'nb1': stk(lambda c: c['node_mlp'][0][1]),
        'nW2': stk(lambda c: c['node_mlp'][1][0]),
        'nb2': stk(lambda c: c['node_mlp'][1][1]),
        'ng': stk(lambda c: c['node_ln'][0]),
        'nbeta': stk(lambda c: c['node_ln'][1]),
    }

    for k in ('Ah', 'Af', 'Bh', 'Bf'):
        cw[k + '2'] = jnp.roll(cw[k], -1, axis=0)

    PA, PB = _project(h_node, nfv, cw['Ah'][0], cw['Af'][0],
                      cw['Bh'][0], cw['Bf'][0])

    def conv_step(carry, w):
        h_node, h_edge1, h_edge2, PA, PB = carry
        G1 = _sc_gather(PA, PB, src1, dst1)
        G2 = _sc_gather(PA, PB, src2, dst2)
        h_edge1 = _edge_update(h_edge1, G1, efv1, w['Ch'], w['Cf'],
                               w['eb1'], w['eW2'], w['eb2'], w['eg'], w['ebeta'])
        aggs1 = _sc_scatter(h_edge1, dst1, zeros)
        h_edge2 = _edge_update(h_edge2, G2, efv2, w['Ch'], w['Cf'],
                               w['eb1'], w['eW2'], w['eb2'], w['eg'], w['ebeta'])
        aggs2 = _sc_scatter(h_edge2, dst2, zeros)
        h_node, PA, PB = _node_update(
            h_node, nfv, aggs1, aggs2, w['Nh'], w['Nf'],
            w['Na'], w['nb1'], w['nW2'], w['nb2'], w['ng'], w['nbeta'],
            w['Ah2'], w['Af2'], w['Bh2'], w['Bf2'])
        return (h_node, h_edge1, h_edge2, PA, PB), None

    (h_node, h_edge1, h_edge2, _, _), _ = lax.scan(
        conv_step, (h_node, h_edge1, h_edge2, PA, PB), cw)

    (Wd1, bd1), (Wd2, bd2) = params['dec_mlp']
    return _decode(h_node, nfv, Wd1[0:H], Wd1[H:H + 1], bd1, Wd2, bd2)


# speedup vs baseline: 4.2909x; 1.0575x over previous
import functools
import jax
import jax.numpy as jnp
import numpy as np
from jax import lax
from jax.experimental import pallas as pl
from jax.experimental.pallas import tpu as pltpu
from jax.experimental.pallas import tpu_sc as plsc

NC = 2
NS = 16
NW = NC * NS

BE = 4000
BN = 2000
KCH = 200
KSC = 192


def _ln_fused(y, g, b):
    m = jnp.mean(y, axis=-1, keepdims=True)
    d = y - m
    v = jnp.mean(d * d, axis=-1, keepdims=True)
    return d * lax.rsqrt(v + 1e-5) * g + b


def _enc_body(x_ref, w1_ref, b1_ref, w2_ref, b2_ref, g_ref, be_ref, o_ref):
    a = jnp.maximum(
        jnp.dot(x_ref[...], w1_ref[...], preferred_element_type=jnp.float32)
        + b1_ref[...], 0.0)
    y = jnp.dot(a, w2_ref[...], preferred_element_type=jnp.float32) + b2_ref[...]
    o_ref[...] = _ln_fused(y, g_ref[...], be_ref[...])


def _encode(xin, W1, b1, W2, b2, g, be, BR):
    R, Din = xin.shape
    H = W2.shape[1]
    return pl.pallas_call(
        _enc_body,
        grid=(R // BR,),
        in_specs=[
            pl.BlockSpec((BR, Din), lambda i: (i, 0)),
            pl.BlockSpec((Din, H), lambda i: (0, 0)),
            pl.BlockSpec((H,), lambda i: (0,)),
            pl.BlockSpec((H, H), lambda i: (0, 0)),
            pl.BlockSpec((H,), lambda i: (0,)),
            pl.BlockSpec((H,), lambda i: (0,)),
            pl.BlockSpec((H,), lambda i: (0,)),
        ],
        out_specs=pl.BlockSpec((BR, H), lambda i: (i, 0)),
        out_shape=jax.ShapeDtypeStruct((R, H), jnp.float32),
    )(xin, W1, b1, W2, b2, g, be)


def _proj_body(h_ref, nfv_ref, ah_ref, af_ref, bh_ref, bf_ref, pa_ref, pb_ref):
    h = h_ref[...]
    nfv = nfv_ref[...]
    pa_ref[...] = jnp.dot(h, ah_ref[...], preferred_element_type=jnp.float32) + nfv * af_ref[...]
    pb_ref[...] = jnp.dot(h, bh_ref[...], preferred_element_type=jnp.float32) + nfv * bf_ref[...]


def _project(h_node, nfv, Ah, Af, Bh, Bf):
    Nn, H = h_node.shape
    out = jax.ShapeDtypeStruct((Nn, H), jnp.float32)
    return pl.pallas_call(
        _proj_body,
        grid=(Nn // BN,),
        in_specs=[
            pl.BlockSpec((BN, H), lambda i: (i, 0)),
            pl.BlockSpec((BN, 1), lambda i: (i, 0)),
            pl.BlockSpec((H, H), lambda i: (0, 0)),
            pl.BlockSpec((1, H), lambda i: (0, 0)),
            pl.BlockSpec((H, H), lambda i: (0, 0)),
            pl.BlockSpec((1, H), lambda i: (0, 0)),
        ],
        out_specs=[
            pl.BlockSpec((BN, H), lambda i: (i, 0)),
            pl.BlockSpec((BN, H), lambda i: (i, 0)),
        ],
        out_shape=[out, out],
    )(h_node, nfv, Ah, Af, Bh, Bf)


def _edge_body(he_ref, ga_ref, efv_ref, ch_ref, cf_ref, b1_ref,
               w2_ref, b2_ref, g_ref, be_ref, o_ref):
    he = he_ref[...]
    z = (ga_ref[...]
         + jnp.dot(he, ch_ref[...], preferred_element_type=jnp.float32)
         + jnp.dot(efv_ref[...], cf_ref[...], preferred_element_type=jnp.float32)
         + b1_ref[...])
    a = jnp.maximum(z, 0.0)
    y = jnp.dot(a, w2_ref[...], preferred_element_type=jnp.float32) + b2_ref[...]
    o_ref[...] = he + _ln_fused(y, g_ref[...], be_ref[...])


def _edge_update(h_edge, G, efv, Ch, Cf, b1, W2, b2, g, be):
    E, H = h_edge.shape
    F = efv.shape[1]
    return pl.pallas_call(
        _edge_body,
        grid=(E // BE,),
        in_specs=[
            pl.BlockSpec((BE, H), lambda i: (i, 0)),
            pl.BlockSpec((BE, H), lambda i: (i, 0)),
            pl.BlockSpec((BE, F), lambda i: (i, 0)),
            pl.BlockSpec((H, H), lambda i: (0, 0)),
            pl.BlockSpec((F, H), lambda i: (0, 0)),
            pl.BlockSpec((H,), lambda i: (0,)),
            pl.BlockSpec((H, H), lambda i: (0, 0)),
            pl.BlockSpec((H,), lambda i: (0,)),
            pl.BlockSpec((H,), lambda i: (0,)),
            pl.BlockSpec((H,), lambda i: (0,)),
        ],
        out_specs=pl.BlockSpec((BE, H), lambda i: (i, 0)),
        out_shape=jax.ShapeDtypeStruct((E, H), jnp.float32),
    )(h_edge, G, efv, Ch, Cf, b1, W2, b2, g, be)


def _node_body(h_ref, nfv_ref, a00_ref, a01_ref, a10_ref, a11_ref,
               nh_ref, nf_ref, na_ref,
               b1_ref, w2_ref, b2_ref, g_ref, be_ref,
               ah_ref, af_ref, bh_ref, bf_ref,
               o_ref, pa_ref, pb_ref):
    h = h_ref[...]
    nfv = nfv_ref[...]
    agg = ((a00_ref[...] + a01_ref[...]) + (a10_ref[...] + a11_ref[...]))
    z = (jnp.dot(h, nh_ref[...], preferred_element_type=jnp.float32)
         + nfv * nf_ref[...]
         + jnp.dot(agg, na_ref[...], preferred_element_type=jnp.float32)
         + b1_ref[...])
    a = jnp.maximum(z, 0.0)
    y = jnp.dot(a, w2_ref[...], preferred_element_type=jnp.float32) + b2_ref[...]
    hn = h + _ln_fused(y, g_ref[...], be_ref[...])
    o_ref[...] = hn
    pa_ref[...] = jnp.dot(hn, ah_ref[...], preferred_element_type=jnp.float32) + nfv * af_ref[...]
    pb_ref[...] = jnp.dot(hn, bh_ref[...], preferred_element_type=jnp.float32) + nfv * bf_ref[...]


def _node_update(h_node, nfv, aggs1, aggs2, Nh, Nf, Na, b1, W2, b2, g, be,
                 Ah2, Af2, Bh2, Bf2):
    Nn, H = h_node.shape
    blk = pl.BlockSpec((BN, H), lambda i: (i, 0))
    wblk = pl.BlockSpec((H, H), lambda i: (0, 0))
    rblk = pl.BlockSpec((1, H), lambda i: (0, 0))
    vblk = pl.BlockSpec((H,), lambda i: (0,))
    out = jax.ShapeDtypeStruct((Nn, H), jnp.float32)
    return pl.pallas_call(
        _node_body,
        grid=(Nn // BN,),
        in_specs=[
            blk,
            pl.BlockSpec((BN, 1), lambda i: (i, 0)),
            blk, blk, blk, blk,
            wblk, rblk, wblk, vblk, wblk, vblk, vblk, vblk,
            wblk, rblk, wblk, rblk,
        ],
        out_specs=[blk, blk, blk],
        out_shape=[out, out, out],
    )(h_node, nfv, aggs1[0], aggs1[1], aggs2[0], aggs2[1],
      Nh, Nf, Na, b1, W2, b2, g, be, Ah2, Af2, Bh2, Bf2)


def _dec_body(h_ref, nfv_ref, w1h_ref, w1f_ref, b1_ref, w2_ref, b2_ref, o_ref):
    z = (jnp.dot(h_ref[...], w1h_ref[...], preferred_element_type=jnp.float32)
         + nfv_ref[...] * w1f_ref[...] + b1_ref[...])
    a = jnp.maximum(z, 0.0)
    o_ref[...] = jnp.dot(a, w2_ref[...], preferred_element_type=jnp.float32) + b2_ref[...]


def _decode(h_node, nfv, W1h, W1f, b1, W2, b2):
    Nn, H = h_node.shape
    D1 = W1h.shape[1]
    DO = W2.shape[1]
    return pl.pallas_call(
        _dec_body,
        grid=(Nn // BN,),
        in_specs=[
            pl.BlockSpec((BN, H), lambda i: (i, 0)),
            pl.BlockSpec((BN, 1), lambda i: (i, 0)),
            pl.BlockSpec((H, D1), lambda i: (0, 0)),
            pl.BlockSpec((1, D1), lambda i: (0, 0)),
            pl.BlockSpec((D1,), lambda i: (0,)),
            pl.BlockSpec((D1, DO), lambda i: (0, 0)),
            pl.BlockSpec((DO,), lambda i: (0,)),
        ],
        out_specs=pl.BlockSpec((BN, DO), lambda i: (i, 0)),
        out_shape=jax.ShapeDtypeStruct((Nn, DO), jnp.float32),
    )(h_node, nfv, W1h, W1f, b1, W2, b2)


def _sc_gather(tableA, tableB, idxA, idxB):
    Erows = idxA.shape[0]
    Hd = tableA.shape[1]
    EW = Erows // NW
    K = KCH
    nchunk = EW // K
    npair = nchunk // 2
    mesh = plsc.VectorSubcoreMesh(core_axis_name="c", subcore_axis_name="s")

    def body(ta_hbm, tb_hbm, idxa_hbm, idxb_hbm, out_hbm,
             idxa_v, idxb_v, a0, b0, a1, b1, sa0, sb0, sa1, sb1):
        wid = lax.axis_index("s") * NC + lax.axis_index("c")
        base = wid * EW
        pltpu.sync_copy(idxa_hbm.at[pl.ds(base, EW)], idxa_v)
        pltpu.sync_copy(idxb_hbm.at[pl.ds(base, EW)], idxb_v)

        def addrows(dst, srcb):
            def row(r, carry):
                for j in range(Hd // 16):
                    s = (r, pl.ds(j * 16, 16))
                    dst[s] = dst[s] + srcb[s]
                return carry
            lax.fori_loop(0, K, row, 0)

        def chunk(c, bufa, bufb, sema, semb):
            ha = pltpu.async_copy(ta_hbm.at[idxa_v.at[pl.ds(c, K)]], bufa, sema)
            hb = pltpu.async_copy(tb_hbm.at[idxb_v.at[pl.ds(c, K)]], bufb, semb)
            return ha, hb

        def finish(c, bufa, bufb, ha, hb):
            ha.wait()
            hb.wait()
            addrows(bufa, bufb)
            pltpu.sync_copy(bufa, out_hbm.at[pl.ds(base + c, K)])

        def pair(g, carry):
            c0 = 2 * g * K
            c1 = c0 + K
            h0 = chunk(c0, a0, b0, sa0, sb0)
            h1 = chunk(c1, a1, b1, sa1, sb1)
            finish(c0, a0, b0, *h0)
            finish(c1, a1, b1, *h1)
            return carry

        lax.fori_loop(0, npair, pair, 0)
        if nchunk % 2 == 1:
            ct = (nchunk - 1) * K
            ht = chunk(ct, a0, b0, sa0, sb0)
            finish(ct, a0, b0, *ht)

    return pl.kernel(
        body,
        out_type=jax.ShapeDtypeStruct((Erows, Hd), jnp.float32),
        mesh=mesh,
        scratch_types=[
            pltpu.VMEM((EW,), jnp.int32),
            pltpu.VMEM((EW,), jnp.int32),
            pltpu.VMEM((K, Hd), jnp.float32),
            pltpu.VMEM((K, Hd), jnp.float32),
            pltpu.VMEM((K, Hd), jnp.float32),
            pltpu.VMEM((K, Hd), jnp.float32),
            pltpu.SemaphoreType.DMA,
            pltpu.SemaphoreType.DMA,
            pltpu.SemaphoreType.DMA,
            pltpu.SemaphoreType.DMA,
        ],
    )(tableA, tableB, idxA, idxB)


def _sc_scatter(rows, idx, zeros):
    Erows, Hd = rows.shape
    Nn = zeros.shape[0]
    EW = Erows // NW
    K = KSC
    nfull = EW // K
    tail = EW - nfull * K
    npair = nfull // 2
    mesh = plsc.VectorSubcoreMesh(core_axis_name="c", subcore_axis_name="s")

    def body(rows_hbm, idx_hbm, zeros_hbm, out_hbm,
             i0, i1, it, r0, r1, rt, accum, s0, s1):
        cid = lax.axis_index("c")
        sid = lax.axis_index("s")
        wid = sid * NC + cid

        @pl.when(sid == 0)
        def _():
            pltpu.sync_copy(zeros_hbm, accum)

        plsc.subcore_barrier()
        base = wid * EW

        def start(c, ibuf, rbuf, sem, n):
            pltpu.sync_copy(idx_hbm.at[pl.ds(base + c, n)], ibuf)
            return pltpu.async_copy(rows_hbm.at[pl.ds(base + c, n)], rbuf, sem)

        def finish(h, ibuf, rbuf):
            h.wait()
            pltpu.sync_copy(rbuf, accum.at[ibuf], add=True)

        def pair(g, carry):
            c0 = 2 * g * K
            c1 = c0 + K
            h0 = start(c0, i0, r0, s0, K)
            h1 = start(c1, i1, r1, s1, K)
            finish(h0, i0, r0)
            finish(h1, i1, r1)
            return carry

        lax.fori_loop(0, npair, pair, 0)
        if nfull % 2 == 1:
            co = (nfull - 1) * K
            ho = start(co, i0, r0, s0, K)
            finish(ho, i0, r0)
        if tail:
            ht = start(nfull * K, it, rt, s1, tail)
            finish(ht, it, rt)
        plsc.subcore_barrier()

        @pl.when(sid == 0)
        def _():
            pltpu.sync_copy(accum, out_hbm.at[cid])

    scratch = [
        pltpu.VMEM((K,), jnp.int32),
        pltpu.VMEM((K,), jnp.int32),
        pltpu.VMEM((max(tail, 8),), jnp.int32),
        pltpu.VMEM((K, Hd), jnp.float32),
        pltpu.VMEM((K, Hd), jnp.float32),
        pltpu.VMEM((max(tail, 8), Hd), jnp.float32),
        pltpu.VMEM_SHARED((Nn, Hd), jnp.float32),
        pltpu.SemaphoreType.DMA,
        pltpu.SemaphoreType.DMA,
    ]
    return pl.kernel(
        body,
        out_type=jax.ShapeDtypeStruct((NC, Nn, Hd), jnp.float32),
        mesh=mesh,
        scratch_types=scratch,
    )(rows, idx, zeros)


def kernel(x, edge_attr, edge_index, node_FVattr, edge_FVattr, params):
    H = 128
    Nn = x.shape[0]
    E = edge_index.shape[1]
    Eh = E // 2
    src1, src2 = edge_index[0, :Eh], edge_index[0, Eh:]
    dst1, dst2 = edge_index[1, :Eh], edge_index[1, Eh:]
    nfv = node_FVattr
    efv1, efv2 = edge_FVattr[:Eh], edge_FVattr[Eh:]
    zeros = jnp.zeros((Nn, H), jnp.float32)

    x_in = jnp.concatenate([x, nfv], axis=1)
    e_in = jnp.concatenate([edge_attr, edge_FVattr], axis=1)
    (We1, be1), (We2, be2) = params['enc_node_mlp']
    gn, bn = params['enc_node_ln']
    h_node = _encode(x_in, We1, be1, We2, be2, gn, bn, BN)
    (Wf1, bf1), (Wf2, bf2) = params['enc_edge_mlp']
    ge, bse = params['enc_edge_ln']
    h_edge1 = _encode(e_in[:Eh], Wf1, bf1, Wf2, bf2, ge, bse, BE)
    h_edge2 = _encode(e_in[Eh:], Wf1, bf1, Wf2, bf2, ge, bse, BE)

    def stk(f):
        return jnp.stack([f(c) for c in params['convs']])

    cw = {
        'Ah': stk(lambda c: c['edge_mlp'][0][0][0:H]),
        'Af': stk(lambda c: c['edge_mlp'][0][0][H:H + 1]),
        'Bh': stk(lambda c: c['edge_mlp'][0][0][H + 1:2 * H + 1]),
        'Bf': stk(lambda c: c['edge_mlp'][0][0][2 * H + 1:2 * H + 2]),
        'Ch': stk(lambda c: c['edge_mlp'][0][0][2 * H + 2:3 * H + 2]),
        'Cf': stk(lambda c: c['edge_mlp'][0][0][3 * H + 2:]),
        'eb1': stk(lambda c: c['edge_mlp'][0][1]),
        'eW2': stk(lambda c: c['edge_mlp'][1][0]),
        'eb2': stk(lambda c: c['edge_mlp'][1][1]),
        'eg': stk(lambda c: c['edge_ln'][0]),
        'ebeta': stk(lambda c: c['edge_ln'][1]),
        'Nh': stk(lambda c: c['node_mlp'][0][0][0:H]),
        'Nf': stk(lambda c: c['node_mlp'][0][0][H:H + 1]),
        'Na': stk(lambda c: c['node_mlp'][0][0][H + 1:]),
        'nb1': stk(lambda c: c['node_mlp'][0][1]),
        'nW2': stk(lambda c: c['node_mlp'][1][0]),
        'nb2': stk(lambda c: c['node_mlp'][1][1]),
        'ng': stk(lambda c: c['node_ln'][0]),
        'nbeta': stk(lambda c: c['node_ln'][1]),
    }

    for k in ('Ah', 'Af', 'Bh', 'Bf'):
        cw[k + '2'] = jnp.roll(cw[k], -1, axis=0)

    PA, PB = _project(h_node, nfv, cw['Ah'][0], cw['Af'][0],
                      cw['Bh'][0], cw['Bf'][0])

    def conv_step(carry, w):
        h_node, h_edge1, h_edge2, PA, PB = carry
        G1 = _sc_gather(PA, PB, src1, dst1)
        G2 = _sc_gather(PA, PB, src2, dst2)
        h_edge1 = _edge_update(h_edge1, G1, efv1, w['Ch'], w['Cf'],
                               w['eb1'], w['eW2'], w['eb2'], w['eg'], w['ebeta'])
        aggs1 = _sc_scatter(h_edge1, dst1, zeros)
        h_edge2 = _edge_update(h_edge2, G2, efv2, w['Ch'], w['Cf'],
                               w['eb1'], w['eW2'], w['eb2'], w['eg'], w['ebeta'])
        aggs2 = _sc_scatter(h_edge2, dst2, zeros)
        h_node, PA, PB = _node_update(
            h_node, nfv, aggs1, aggs2, w['Nh'], w['Nf'],
            w['Na'], w['nb1'], w['nW2'], w['nb2'], w['ng'], w['nbeta'],
            w['Ah2'], w['Af2'], w['Bh2'], w['Bf2'])
        return (h_node, h_edge1, h_edge2, PA, PB), None

    (h_node, h_edge1, h_edge2, _, _), _ = lax.scan(
        conv_step, (h_node, h_edge1, h_edge2, PA, PB), cw)

    (Wd1, bd1), (Wd2, bd2) = params['dec_mlp']
    return _decode(h_node, nfv, Wd1[0:H], Wd1[H:H + 1], bd1, Wd2, bd2)


# speedup vs baseline: 4.3564x; 1.0153x over previous
import functools
import jax
import jax.numpy as jnp
import numpy as np
from jax import lax
from jax.experimental import pallas as pl
from jax.experimental.pallas import tpu as pltpu
from jax.experimental.pallas import tpu_sc as plsc

NC = 2
NS = 16
NW = NC * NS

BE = 8000
BN = 2000
KCH = 200
KSC = 192


def _ln_fused(y, g, b):
    m = jnp.mean(y, axis=-1, keepdims=True)
    d = y - m
    v = jnp.mean(d * d, axis=-1, keepdims=True)
    return d * lax.rsqrt(v + 1e-5) * g + b


def _enc_body(x_ref, w1_ref, b1_ref, w2_ref, b2_ref, g_ref, be_ref, o_ref):
    a = jnp.maximum(
        jnp.dot(x_ref[...], w1_ref[...], preferred_element_type=jnp.float32)
        + b1_ref[...], 0.0)
    y = jnp.dot(a, w2_ref[...], preferred_element_type=jnp.float32) + b2_ref[...]
    o_ref[...] = _ln_fused(y, g_ref[...], be_ref[...])


def _encode(xin, W1, b1, W2, b2, g, be, BR):
    R, Din = xin.shape
    H = W2.shape[1]
    return pl.pallas_call(
        _enc_body,
        grid=(R // BR,),
        in_specs=[
            pl.BlockSpec((BR, Din), lambda i: (i, 0)),
            pl.BlockSpec((Din, H), lambda i: (0, 0)),
            pl.BlockSpec((H,), lambda i: (0,)),
            pl.BlockSpec((H, H), lambda i: (0, 0)),
            pl.BlockSpec((H,), lambda i: (0,)),
            pl.BlockSpec((H,), lambda i: (0,)),
            pl.BlockSpec((H,), lambda i: (0,)),
        ],
        out_specs=pl.BlockSpec((BR, H), lambda i: (i, 0)),
        out_shape=jax.ShapeDtypeStruct((R, H), jnp.float32),
    )(xin, W1, b1, W2, b2, g, be)


def _proj_body(h_ref, nfv_ref, ah_ref, af_ref, bh_ref, bf_ref, pa_ref, pb_ref):
    h = h_ref[...]
    nfv = nfv_ref[...]
    pa_ref[...] = jnp.dot(h, ah_ref[...], preferred_element_type=jnp.float32) + nfv * af_ref[...]
    pb_ref[...] = jnp.dot(h, bh_ref[...], preferred_element_type=jnp.float32) + nfv * bf_ref[...]


def _project(h_node, nfv, Ah, Af, Bh, Bf):
    Nn, H = h_node.shape
    out = jax.ShapeDtypeStruct((Nn, H), jnp.float32)
    return pl.pallas_call(
        _proj_body,
        grid=(Nn // BN,),
        in_specs=[
            pl.BlockSpec((BN, H), lambda i: (i, 0)),
            pl.BlockSpec((BN, 1), lambda i: (i, 0)),
            pl.BlockSpec((H, H), lambda i: (0, 0)),
            pl.BlockSpec((1, H), lambda i: (0, 0)),
            pl.BlockSpec((H, H), lambda i: (0, 0)),
            pl.BlockSpec((1, H), lambda i: (0, 0)),
        ],
        out_specs=[
            pl.BlockSpec((BN, H), lambda i: (i, 0)),
            pl.BlockSpec((BN, H), lambda i: (i, 0)),
        ],
        out_shape=[out, out],
    )(h_node, nfv, Ah, Af, Bh, Bf)


def _edge_body(he_ref, ga_ref, efv_ref, ch_ref, cf_ref, b1_ref,
               w2_ref, b2_ref, g_ref, be_ref, o_ref):
    he = he_ref[...]
    z = (ga_ref[...]
         + jnp.dot(he, ch_ref[...], preferred_element_type=jnp.float32)
         + jnp.dot(efv_ref[...], cf_ref[...], preferred_element_type=jnp.float32)
         + b1_ref[...])
    a = jnp.maximum(z, 0.0)
    y = jnp.dot(a, w2_ref[...], preferred_element_type=jnp.float32) + b2_ref[...]
    o_ref[...] = he + _ln_fused(y, g_ref[...], be_ref[...])


def _edge_update(h_edge, G, efv, Ch, Cf, b1, W2, b2, g, be):
    E, H = h_edge.shape
    F = efv.shape[1]
    return pl.pallas_call(
        _edge_body,
        grid=(E // BE,),
        in_specs=[
            pl.BlockSpec((BE, H), lambda i: (i, 0)),
            pl.BlockSpec((BE, H), lambda i: (i, 0)),
            pl.BlockSpec((BE, F), lambda i: (i, 0)),
            pl.BlockSpec((H, H), lambda i: (0, 0)),
            pl.BlockSpec((F, H), lambda i: (0, 0)),
            pl.BlockSpec((H,), lambda i: (0,)),
            pl.BlockSpec((H, H), lambda i: (0, 0)),
            pl.BlockSpec((H,), lambda i: (0,)),
            pl.BlockSpec((H,), lambda i: (0,)),
            pl.BlockSpec((H,), lambda i: (0,)),
        ],
        out_specs=pl.BlockSpec((BE, H), lambda i: (i, 0)),
        out_shape=jax.ShapeDtypeStruct((E, H), jnp.float32),
    )(h_edge, G, efv, Ch, Cf, b1, W2, b2, g, be)


def _node_body(h_ref, nfv_ref, a00_ref, a01_ref, a10_ref, a11_ref,
               nh_ref, nf_ref, na_ref,
               b1_ref, w2_ref, b2_ref, g_ref, be_ref,
               ah_ref, af_ref, bh_ref, bf_ref,
               o_ref, pa_ref, pb_ref):
    h = h_ref[...]
    nfv = nfv_ref[...]
    agg = ((a00_ref[...] + a01_ref[...]) + (a10_ref[...] + a11_ref[...]))
    z = (jnp.dot(h, nh_ref[...], preferred_element_type=jnp.float32)
         + nfv * nf_ref[...]
         + jnp.dot(agg, na_ref[...], preferred_element_type=jnp.float32)
         + b1_ref[...])
    a = jnp.maximum(z, 0.0)
    y = jnp.dot(a, w2_ref[...], preferred_element_type=jnp.float32) + b2_ref[...]
    hn = h + _ln_fused(y, g_ref[...], be_ref[...])
    o_ref[...] = hn
    pa_ref[...] = jnp.dot(hn, ah_ref[...], preferred_element_type=jnp.float32) + nfv * af_ref[...]
    pb_ref[...] = jnp.dot(hn, bh_ref[...], preferred_element_type=jnp.float32) + nfv * bf_ref[...]


def _node_update(h_node, nfv, aggs1, aggs2, Nh, Nf, Na, b1, W2, b2, g, be,
                 Ah2, Af2, Bh2, Bf2):
    Nn, H = h_node.shape
    blk = pl.BlockSpec((BN, H), lambda i: (i, 0))
    wblk = pl.BlockSpec((H, H), lambda i: (0, 0))
    rblk = pl.BlockSpec((1, H), lambda i: (0, 0))
    vblk = pl.BlockSpec((H,), lambda i: (0,))
    out = jax.ShapeDtypeStruct((Nn, H), jnp.float32)
    return pl.pallas_call(
        _node_body,
        grid=(Nn // BN,),
        in_specs=[
            blk,
            pl.BlockSpec((BN, 1), lambda i: (i, 0)),
            blk, blk, blk, blk,
            wblk, rblk, wblk, vblk, wblk, vblk, vblk, vblk,
            wblk, rblk, wblk, rblk,
        ],
        out_specs=[blk, blk, blk],
        out_shape=[out, out, out],
    )(h_node, nfv, aggs1[0], aggs1[1], aggs2[0], aggs2[1],
      Nh, Nf, Na, b1, W2, b2, g, be, Ah2, Af2, Bh2, Bf2)


def _dec_body(h_ref, nfv_ref, w1h_ref, w1f_ref, b1_ref, w2_ref, b2_ref, o_ref):
    z = (jnp.dot(h_ref[...], w1h_ref[...], preferred_element_type=jnp.float32)
         + nfv_ref[...] * w1f_ref[...] + b1_ref[...])
    a = jnp.maximum(z, 0.0)
    o_ref[...] = jnp.dot(a, w2_ref[...], preferred_element_type=jnp.float32) + b2_ref[...]


def _decode(h_node, nfv, W1h, W1f, b1, W2, b2):
    Nn, H = h_node.shape
    D1 = W1h.shape[1]
    DO = W2.shape[1]
    return pl.pallas_call(
        _dec_body,
        grid=(Nn // BN,),
        in_specs=[
            pl.BlockSpec((BN, H), lambda i: (i, 0)),
            pl.BlockSpec((BN, 1), lambda i: (i, 0)),
            pl.BlockSpec((H, D1), lambda i: (0, 0)),
            pl.BlockSpec((1, D1), lambda i: (0, 0)),
            pl.BlockSpec((D1,), lambda i: (0,)),
            pl.BlockSpec((D1, DO), lambda i: (0, 0)),
            pl.BlockSpec((DO,), lambda i: (0,)),
        ],
        out_specs=pl.BlockSpec((BN, DO), lambda i: (i, 0)),
        out_shape=jax.ShapeDtypeStruct((Nn, DO), jnp.float32),
    )(h_node, nfv, W1h, W1f, b1, W2, b2)


def _sc_gather(tableA, tableB, idxA, idxB):
    Erows = idxA.shape[0]
    Hd = tableA.shape[1]
    EW = Erows // NW
    K = KCH
    nchunk = EW // K
    npair = nchunk // 2
    mesh = plsc.VectorSubcoreMesh(core_axis_name="c", subcore_axis_name="s")

    def body(ta_hbm, tb_hbm, idxa_hbm, idxb_hbm, out_hbm,
             idxa_v, idxb_v, a0, b0, a1, b1, sa0, sb0, sa1, sb1):
        wid = lax.axis_index("s") * NC + lax.axis_index("c")
        base = wid * EW
        pltpu.sync_copy(idxa_hbm.at[pl.ds(base, EW)], idxa_v)
        pltpu.sync_copy(idxb_hbm.at[pl.ds(base, EW)], idxb_v)

        def addrows(dst, srcb):
            def row(r, carry):
                for j in range(Hd // 16):
                    s = (r, pl.ds(j * 16, 16))
                    dst[s] = dst[s] + srcb[s]
                return carry
            lax.fori_loop(0, K, row, 0)

        def chunk(c, bufa, bufb, sema, semb):
            ha = pltpu.async_copy(ta_hbm.at[idxa_v.at[pl.ds(c, K)]], bufa, sema)
            hb = pltpu.async_copy(tb_hbm.at[idxb_v.at[pl.ds(c, K)]], bufb, semb)
            return ha, hb

        def finish(c, bufa, bufb, ha, hb):
            ha.wait()
            hb.wait()
            addrows(bufa, bufb)
            pltpu.sync_copy(bufa, out_hbm.at[pl.ds(base + c, K)])

        def pair(g, carry):
            c0 = 2 * g * K
            c1 = c0 + K
            h0 = chunk(c0, a0, b0, sa0, sb0)
            h1 = chunk(c1, a1, b1, sa1, sb1)
            finish(c0, a0, b0, *h0)
            finish(c1, a1, b1, *h1)
            return carry

        lax.fori_loop(0, npair, pair, 0)
        if nchunk % 2 == 1:
            ct = (nchunk - 1) * K
            ht = chunk(ct, a0, b0, sa0, sb0)
            finish(ct, a0, b0, *ht)

    return pl.kernel(
        body,
        out_type=jax.ShapeDtypeStruct((Erows, Hd), jnp.float32),
        mesh=mesh,
        scratch_types=[
            pltpu.VMEM((EW,), jnp.int32),
            pltpu.VMEM((EW,), jnp.int32),
            pltpu.VMEM((K, Hd), jnp.float32),
            pltpu.VMEM((K, Hd), jnp.float32),
            pltpu.VMEM((K, Hd), jnp.float32),
            pltpu.VMEM((K, Hd), jnp.float32),
            pltpu.SemaphoreType.DMA,
            pltpu.SemaphoreType.DMA,
            pltpu.SemaphoreType.DMA,
            pltpu.SemaphoreType.DMA,
        ],
    )(tableA, tableB, idxA, idxB)


def _sc_scatter(rows, idx, zeros):
    Erows, Hd = rows.shape
    Nn = zeros.shape[0]
    EW = Erows // NW
    K = KSC
    nfull = EW // K
    tail = EW - nfull * K
    npair = nfull // 2
    mesh = plsc.VectorSubcoreMesh(core_axis_name="c", subcore_axis_name="s")

    def body(rows_hbm, idx_hbm, zeros_hbm, out_hbm,
             i0, i1, it, r0, r1, rt, accum, s0, s1):
        cid = lax.axis_index("c")
        sid = lax.axis_index("s")
        wid = sid * NC + cid

        @pl.when(sid == 0)
        def _():
            pltpu.sync_copy(zeros_hbm, accum)

        plsc.subcore_barrier()
        base = wid * EW

        def start(c, ibuf, rbuf, sem, n):
            pltpu.sync_copy(idx_hbm.at[pl.ds(base + c, n)], ibuf)
            return pltpu.async_copy(rows_hbm.at[pl.ds(base + c, n)], rbuf, sem)

        def finish(h, ibuf, rbuf):
            h.wait()
            pltpu.sync_copy(rbuf, accum.at[ibuf], add=True)

        def pair(g, carry):
            c0 = 2 * g * K
            c1 = c0 + K
            h0 = start(c0, i0, r0, s0, K)
            h1 = start(c1, i1, r1, s1, K)
            finish(h0, i0, r0)
            finish(h1, i1, r1)
            return carry

        lax.fori_loop(0, npair, pair, 0)
        if nfull % 2 == 1:
            co = (nfull - 1) * K
            ho = start(co, i0, r0, s0, K)
            finish(ho, i0, r0)
        if tail:
            ht = start(nfull * K, it, rt, s1, tail)
            finish(ht, it, rt)
        plsc.subcore_barrier()

        @pl.when(sid == 0)
        def _():
            pltpu.sync_copy(accum, out_hbm.at[cid])

    scratch = [
        pltpu.VMEM((K,), jnp.int32),
        pltpu.VMEM((K,), jnp.int32),
        pltpu.VMEM((max(tail, 8),), jnp.int32),
        pltpu.VMEM((K, Hd), jnp.float32),
        pltpu.VMEM((K, Hd), jnp.float32),
        pltpu.VMEM((max(tail, 8), Hd), jnp.float32),
        pltpu.VMEM_SHARED((Nn, Hd), jnp.float32),
        pltpu.SemaphoreType.DMA,
        pltpu.SemaphoreType.DMA,
    ]
    return pl.kernel(
        body,
        out_type=jax.ShapeDtypeStruct((NC, Nn, Hd), jnp.float32),
        mesh=mesh,
        scratch_types=scratch,
    )(rows, idx, zeros)


def kernel(x, edge_attr, edge_index, node_FVattr, edge_FVattr, params):
    H = 128
    Nn = x.shape[0]
    E = edge_index.shape[1]
    Eh = E // 2
    src1, src2 = edge_index[0, :Eh], edge_index[0, Eh:]
    dst1, dst2 = edge_index[1, :Eh], edge_index[1, Eh:]
    nfv = node_FVattr
    efv1, efv2 = edge_FVattr[:Eh], edge_FVattr[Eh:]
    zeros = jnp.zeros((Nn, H), jnp.float32)

    x_in = jnp.concatenate([x, nfv], axis=1)
    e_in = jnp.concatenate([edge_attr, edge_FVattr], axis=1)
    (We1, be1), (We2, be2) = params['enc_node_mlp']
    gn, bn = params['enc_node_ln']
    h_node = _encode(x_in, We1, be1, We2, be2, gn, bn, BN)
    (Wf1, bf1), (Wf2, bf2) = params['enc_edge_mlp']
    ge, bse = params['enc_edge_ln']
    h_edge1 = _encode(e_in[:Eh], Wf1, bf1, Wf2, bf2, ge, bse, BE)
    h_edge2 = _encode(e_in[Eh:], Wf1, bf1, Wf2, bf2, ge, bse, BE)

    def stk(f):
        return jnp.stack([f(c) for c in params['convs']])

    cw = {
        'Ah': stk(lambda c: c['edge_mlp'][0][0][0:H]),
        'Af': stk(lambda c: c['edge_mlp'][0][0][H:H + 1]),
        'Bh': stk(lambda c: c['edge_mlp'][0][0][H + 1:2 * H + 1]),
        'Bf': stk(lambda c: c['edge_mlp'][0][0][2 * H + 1:2 * H + 2]),
        'Ch': stk(lambda c: c['edge_mlp'][0][0][2 * H + 2:3 * H + 2]),
        'Cf': stk(lambda c: c['edge_mlp'][0][0][3 * H + 2:]),
        'eb1': stk(lambda c: c['edge_mlp'][0][1]),
        'eW2': stk(lambda c: c['edge_mlp'][1][0]),
        'eb2': stk(lambda c: c['edge_mlp'][1][1]),
        'eg': stk(lambda c: c['edge_ln'][0]),
        'ebeta': stk(lambda c: c['edge_ln'][1]),
        'Nh': stk(lambda c: c['node_mlp'][0][0][0:H]),
        'Nf': stk(lambda c: c['node_mlp'][0][0][H:H + 1]),
        'Na': stk(lambda c: c['node_mlp'][0][0][H + 1:]),
        'nb1': stk(lambda c: c['node_mlp'][0][1]),
        'nW2': stk(lambda c: c['node_mlp'][1][0]),
        'nb2': stk(lambda c: c['node_mlp'][1][1]),
        'ng': stk(lambda c: c['node_ln'][0]),
        'nbeta': stk(lambda c: c['node_ln'][1]),
    }

    for k in ('Ah', 'Af', 'Bh', 'Bf'):
        cw[k + '2'] = jnp.roll(cw[k], -1, axis=0)

    PA, PB = _project(h_node, nfv, cw['Ah'][0], cw['Af'][0],
                      cw['Bh'][0], cw['Bf'][0])

    def conv_step(carry, w):
        h_node, h_edge1, h_edge2, PA, PB = carry
        G1 = _sc_gather(PA, PB, src1, dst1)
        G2 = _sc_gather(PA, PB, src2, dst2)
        h_edge1 = _edge_update(h_edge1, G1, efv1, w['Ch'], w['Cf'],
                               w['eb1'], w['eW2'], w['eb2'], w['eg'], w['ebeta'])
        aggs1 = _sc_scatter(h_edge1, dst1, zeros)
        h_edge2 = _edge_update(h_edge2, G2, efv2, w['Ch'], w['Cf'],
                               w['eb1'], w['eW2'], w['eb2'], w['eg'], w['ebeta'])
        aggs2 = _sc_scatter(h_edge2, dst2, zeros)
        h_node, PA, PB = _node_update(
            h_node, nfv, aggs1, aggs2, w['Nh'], w['Nf'],
            w['Na'], w['nb1'], w['nW2'], w['nb2'], w['ng'], w['nbeta'],
            w['Ah2'], w['Af2'], w['Bh2'], w['Bf2'])
        return (h_node, h_edge1, h_edge2, PA, PB), None

    (h_node, h_edge1, h_edge2, _, _), _ = lax.scan(
        conv_step, (h_node, h_edge1, h_edge2, PA, PB), cw)

    (Wd1, bd1), (Wd2, bd2) = params['dec_mlp']
    return _decode(h_node, nfv, Wd1[0:H], Wd1[H:H + 1], bd1, Wd2, bd2)
